# Initial kernel scaffold; baseline (speedup 1.0000x reference)
import jax
import jax.numpy as jnp
from jax.experimental import pallas as pl


def kernel(x, edge_index, edge_attr, batch, params):
    raise NotImplementedError("write your pallas kernel here")


# speedup vs baseline: 2.5071x; 2.5071x over previous
import functools

import jax
import jax.numpy as jnp
from jax import lax
from jax.experimental import pallas as pl
from jax.experimental.pallas import tpu as pltpu

N = 10000
E = 320000
HID = 64
B = 64
GROUPS = 10
LAMDA = 0.01
EPS = 1e-5
NEG = -1e30

_NODE_BLK = 400
_EDGE_BLK = 2000


def _silu(v):
    return v * jax.nn.sigmoid(v)


def _elu(v):
    return jnp.where(v > 0, v, jnp.exp(jnp.minimum(v, 0.0)) - 1.0)


def _pad8(v):
    return jnp.broadcast_to(v[None, :], (8, v.shape[0]))


def _lin_body(act, x_ref, wt_ref, b_ref, o_ref):
    v = jnp.dot(x_ref[...], wt_ref[...], preferred_element_type=jnp.float32)
    v = v + b_ref[0:1, :]
    if act == "silu":
        v = _silu(v)
    o_ref[...] = v


def _linear(x, W, b, act, blk):
    rows, din = x.shape
    h = W.shape[0]
    grid = rows // blk
    return pl.pallas_call(
        functools.partial(_lin_body, act),
        grid=(grid,),
        in_specs=[
            pl.BlockSpec((blk, din), lambda i: (i, 0)),
            pl.BlockSpec((din, h), lambda i: (0, 0)),
            pl.BlockSpec((8, h), lambda i: (0, 0)),
        ],
        out_specs=pl.BlockSpec((blk, h), lambda i: (i, 0)),
        out_shape=jax.ShapeDtypeStruct((rows, h), jnp.float32),
    )(x, W.T, _pad8(b))


def _gru_math(g, hprev, wr, wz, wn, ur, uz, un, br, bz, bn, cr, cz, cn):
    ir = jnp.dot(g, wr, preferred_element_type=jnp.float32) + br[0:1, :]
    iz = jnp.dot(g, wz, preferred_element_type=jnp.float32) + bz[0:1, :]
    inn = jnp.dot(g, wn, preferred_element_type=jnp.float32) + bn[0:1, :]
    hr = jnp.dot(hprev, ur, preferred_element_type=jnp.float32) + cr[0:1, :]
    hz = jnp.dot(hprev, uz, preferred_element_type=jnp.float32) + cz[0:1, :]
    hn = jnp.dot(hprev, un, preferred_element_type=jnp.float32) + cn[0:1, :]
    r = jax.nn.sigmoid(ir + hr)
    z = jax.nn.sigmoid(iz + hz)
    n = jnp.tanh(inn + r * hn)
    return jnp.maximum((1.0 - z) * n + z * hprev, 0.0)


def _gru_node_body(num0_ref, num1_ref, den0_ref, den1_ref, bias_ref, x_ref,
                   wr, wz, wn, ur, uz, un, br, bz, bn, cr, cz, cn, o_ref):
    num = num0_ref[...] + num1_ref[...]
    den = den0_ref[...][:, 0:1] + den1_ref[...][:, 0:1]
    g = _elu(num / (den + 1e-16) + bias_ref[0:1, :])
    o_ref[...] = _gru_math(g, x_ref[...], wr[...], wz[...], wn[...],
                           ur[...], uz[...], un[...], br[...], bz[...],
                           bn[...], cr[...], cz[...], cn[...])


def _split_gru(p):
    Wih, Whh = p["W_ih"], p["W_hh"]
    bih, bhh = p["b_ih"], p["b_hh"]
    outs = []
    for i in range(3):
        outs.append(Wih[i * HID:(i + 1) * HID].T)
    for i in range(3):
        outs.append(Whh[i * HID:(i + 1) * HID].T)
    for i in range(3):
        outs.append(_pad8(bih[i * HID:(i + 1) * HID]))
    for i in range(3):
        outs.append(_pad8(bhh[i * HID:(i + 1) * HID]))
    return outs


def _gru_node(num0, num1, den0, den1, bias, x, gp):
    blk = _NODE_BLK
    grid = N // blk
    wmats = _split_gru(gp)
    full = lambda s: pl.BlockSpec(s, lambda i: (0, 0))
    rowspec = lambda w: pl.BlockSpec((blk, w), lambda i: (i, 0))
    return pl.pallas_call(
        _gru_node_body,
        grid=(grid,),
        in_specs=[rowspec(HID), rowspec(HID), rowspec(16), rowspec(16),
                  full((8, HID)), rowspec(HID)]
                 + [full((HID, HID))] * 6 + [full((8, HID))] * 6,
        out_specs=rowspec(HID),
        out_shape=jax.ShapeDtypeStruct((N, HID), jnp.float32),
    )(num0, num1, den0, den1, _pad8(bias), x, *wmats)


def _gru_graph_body(num_ref, den_ref, bias_ref, h_ref,
                    wr, wz, wn, ur, uz, un, br, bz, bn, cr, cz, cn, o_ref):
    g = _elu(num_ref[...] / (den_ref[...][:, 0:1] + 1e-16) + bias_ref[0:1, :])
    o_ref[...] = _gru_math(g, h_ref[...], wr[...], wz[...], wn[...],
                           ur[...], uz[...], un[...], br[...], bz[...],
                           bn[...], cr[...], cz[...], cn[...])


def _gru_graph(num, den, bias, h, gp):
    wmats = _split_gru(gp)
    return pl.pallas_call(
        _gru_graph_body,
        out_shape=jax.ShapeDtypeStruct((B, HID), jnp.float32),
    )(num, den, _pad8(bias), h, *wmats)


def _softmax_s(xb, wnt, bn16):
    logits = jnp.dot(xb, wnt, preferred_element_type=jnp.float32) + bn16[0:1, :]
    m = jnp.max(logits, axis=1, keepdims=True)
    s = jnp.exp(logits - m)
    return s / jnp.sum(s, axis=1, keepdims=True)


def _dgn_stats_body(x_ref, wnt_ref, bn_ref, m_ref, q_ref):
    i = pl.program_id(0)
    xb = x_ref[...]
    s = _softmax_s(xb, wnt_ref[...], bn_ref[...])
    mp = lax.dot_general(s, xb, (((0,), (0,)), ((), ())),
                         preferred_element_type=jnp.float32)
    qp = lax.dot_general(s * s, xb * xb, (((0,), (0,)), ((), ())),
                         preferred_element_type=jnp.float32)

    @pl.when(i == 0)
    def _():
        m_ref[...] = jnp.zeros_like(m_ref)
        q_ref[...] = jnp.zeros_like(q_ref)

    m_ref[...] += mp
    q_ref[...] += qp


def _dgn_apply_body(nrows, x_ref, wnt_ref, bn_ref, m_ref, q_ref,
                    bw_ref, bb_ref, o_ref):
    xb = x_ref[...]
    s = _softmax_s(xb, wnt_ref[...], bn_ref[...])
    mean = m_ref[...] * (1.0 / nrows)
    var = q_ref[...] * (1.0 / nrows) - mean * mean
    rstd = lax.rsqrt(var + EPS)
    wsc = bw_ref[...] * rstd
    a = jnp.dot(s, wsc, preferred_element_type=jnp.float32)
    c = jnp.sum(mean * wsc - bb_ref[...], axis=0, keepdims=True)
    o_ref[...] = xb + LAMDA * (xb * a - c)


def _dgn_prep(p):
    W, b = p["lin"]["W"], p["lin"]["b"]
    wnt = jnp.zeros((HID, 16), jnp.float32).at[:, :GROUPS].set(W.T)
    bn16 = jnp.full((16,), NEG, jnp.float32).at[:GROUPS].set(b)
    bw = jnp.zeros((16, HID), jnp.float32).at[:GROUPS].set(
        p["bn_w"].reshape(GROUPS, HID))
    bb = jnp.zeros((16, HID), jnp.float32).at[:GROUPS].set(
        p["bn_b"].reshape(GROUPS, HID))
    return wnt, _pad8(bn16), bw, bb


def _dgn_node(x, p):
    wnt, bn16, bw, bb = _dgn_prep(p)
    blk = _NODE_BLK
    grid = N // blk
    full = lambda s: pl.BlockSpec(s, lambda i: (0, 0))
    m, q = pl.pallas_call(
        _dgn_stats_body,
        grid=(grid,),
        in_specs=[pl.BlockSpec((blk, HID), lambda i: (i, 0)),
                  full((HID, 16)), full((8, 16))],
        out_specs=[full((16, HID)), full((16, HID))],
        out_shape=[jax.ShapeDtypeStruct((16, HID), jnp.float32)] * 2,
    )(x, wnt, bn16)
    return pl.pallas_call(
        functools.partial(_dgn_apply_body, float(N)),
        grid=(grid,),
        in_specs=[pl.BlockSpec((blk, HID), lambda i: (i, 0)),
                  full((HID, 16)), full((8, 16)), full((16, HID)),
                  full((16, HID)), full((16, HID)), full((16, HID))],
        out_specs=pl.BlockSpec((blk, HID), lambda i: (i, 0)),
        out_shape=jax.ShapeDtypeStruct((N, HID), jnp.float32),
    )(x, wnt, bn16, m, q, bw, bb)


def _dgn_graph_body(x_ref, wnt_ref, bn_ref, bw_ref, bb_ref, o_ref):
    xb = x_ref[...]
    s = _softmax_s(xb, wnt_ref[...], bn_ref[...])
    mp = lax.dot_general(s, xb, (((0,), (0,)), ((), ())),
                         preferred_element_type=jnp.float32)
    qp = lax.dot_general(s * s, xb * xb, (((0,), (0,)), ((), ())),
                         preferred_element_type=jnp.float32)
    mean = mp * (1.0 / B)
    var = qp * (1.0 / B) - mean * mean
    rstd = lax.rsqrt(var + EPS)
    wsc = bw_ref[...] * rstd
    a = jnp.dot(s, wsc, preferred_element_type=jnp.float32)
    c = jnp.sum(mean * wsc - bb_ref[...], axis=0, keepdims=True)
    o_ref[...] = xb + LAMDA * (xb * a - c)


def _dgn_graph(x, p):
    wnt, bn16, bw, bb = _dgn_prep(p)
    return pl.pallas_call(
        _dgn_graph_body,
        out_shape=jax.ShapeDtypeStruct((B, HID), jnp.float32),
    )(x, wnt, bn16, bw, bb)


def _pool_body(nblocks, bf_ref, x_ref, o_ref):
    i = pl.program_id(0)
    iota = lax.broadcasted_iota(jnp.int32, (1, B), 1).astype(jnp.float32)
    onehot = (bf_ref[...][:, 0:1] == iota).astype(jnp.float32)
    part = lax.dot_general(onehot, x_ref[...], (((0,), (0,)), ((), ())),
                           preferred_element_type=jnp.float32)

    @pl.when(i == 0)
    def _():
        o_ref[...] = jnp.zeros_like(o_ref)

    o_ref[...] += part

    @pl.when(i == nblocks - 1)
    def _():
        o_ref[...] = jnp.maximum(o_ref[...], 0.0)


def _pool(x, batchf):
    blk = _NODE_BLK
    grid = N // blk
    return pl.pallas_call(
        functools.partial(_pool_body, grid),
        grid=(grid,),
        in_specs=[pl.BlockSpec((blk, 8), lambda i: (i, 0)),
                  pl.BlockSpec((blk, HID), lambda i: (i, 0))],
        out_specs=pl.BlockSpec((B, HID), lambda i: (0, 0)),
        out_shape=jax.ShapeDtypeStruct((B, HID), jnp.float32),
    )(batchf, x)


def _ggat_alpha(xlb, bf, xr, att):
    iota = lax.broadcasted_iota(jnp.int32, (1, B), 1).astype(jnp.float32)
    onehot = (bf[:, 0:1] == iota).astype(jnp.float32)
    e = xlb + jnp.dot(onehot, xr, preferred_element_type=jnp.float32)
    e = jnp.where(e > 0, e, 0.01 * e)
    alpha = jnp.dot(e, att, preferred_element_type=jnp.float32)
    return onehot, alpha


def _ggat1_body(nblocks, xl_ref, bf_ref, xr_ref, att_ref, amax_ref):
    i = pl.program_id(0)
    onehot, alpha = _ggat_alpha(xl_ref[...], bf_ref[...], xr_ref[...],
                                att_ref[...][:, 0:1])
    masked = jnp.where(onehot > 0, alpha, NEG)
    pmax = jnp.max(masked, axis=0, keepdims=True)

    @pl.when(i == 0)
    def _():
        amax_ref[...] = jnp.full_like(amax_ref, NEG)

    amax_ref[...] = jnp.maximum(amax_ref[...], jnp.broadcast_to(pmax, (8, B)))


def _ggat2_body(xl_ref, bf_ref, xr_ref, att_ref, amax_ref, num_ref, den_ref):
    i = pl.program_id(0)
    onehot, alpha = _ggat_alpha(xl_ref[...], bf_ref[...], xr_ref[...],
                                att_ref[...][:, 0:1])
    am = amax_ref[...][0:1, :]
    am = jnp.where(am < -1e29, 0.0, am)
    amrow = jnp.sum(onehot * am, axis=1, keepdims=True)
    ex = jnp.exp(alpha - amrow)
    np_ = lax.dot_general(onehot, ex * xl_ref[...], (((0,), (0,)), ((), ())),
                          preferred_element_type=jnp.float32)
    dp = lax.dot_general(onehot, jnp.broadcast_to(ex, ex.shape[:1] + (8,)),
                         (((0,), (0,)), ((), ())),
                         preferred_element_type=jnp.float32)

    @pl.when(i == 0)
    def _():
        num_ref[...] = jnp.zeros_like(num_ref)
        den_ref[...] = jnp.zeros_like(den_ref)

    num_ref[...] += np_
    den_ref[...] += dp


def _ggat(xl, batchf, xr, att):
    blk = _NODE_BLK
    grid = N // blk
    full = lambda s: pl.BlockSpec(s, lambda i: (0, 0))
    att2 = jnp.broadcast_to(att[:, None], (HID, 8))
    amax = pl.pallas_call(
        functools.partial(_ggat1_body, grid),
        grid=(grid,),
        in_specs=[pl.BlockSpec((blk, HID), lambda i: (i, 0)),
                  pl.BlockSpec((blk, 8), lambda i: (i, 0)),
                  full((B, HID)), full((HID, 8))],
        out_specs=full((8, B)),
        out_shape=jax.ShapeDtypeStruct((8, B), jnp.float32),
    )(xl, batchf, xr, att2)
    num, den = pl.pallas_call(
        _ggat2_body,
        grid=(grid,),
        in_specs=[pl.BlockSpec((blk, HID), lambda i: (i, 0)),
                  pl.BlockSpec((blk, 8), lambda i: (i, 0)),
                  full((B, HID)), full((HID, 8)), full((8, B))],
        out_specs=[full((B, HID)), full((B, 8))],
        out_shape=[jax.ShapeDtypeStruct((B, HID), jnp.float32),
                   jax.ShapeDtypeStruct((B, 8), jnp.float32)],
    )(xl, batchf, xr, att2, amax)
    return num, den


def _edge_phase(xl, xr, se, src, dst, att):
    e = xl[src] + xr[dst] + se
    e = jnp.where(e > 0, e, 0.01 * e)
    alpha = e @ att
    gmax = jnp.max(alpha)
    ex = jnp.exp(alpha - gmax)
    num = jax.ops.segment_sum(ex[:, None] * xl[src], dst, num_segments=N)
    den = jax.ops.segment_sum(ex, dst, num_segments=N)
    z16 = jnp.zeros((N, 16), jnp.float32)
    den16 = z16.at[:, 0].set(den)
    return num, jnp.zeros_like(num), den16, jnp.zeros_like(den16)


def kernel(x, edge_index, edge_attr, batch, params):
    src, dst = edge_index[0], edge_index[1]
    batchf = jnp.broadcast_to(batch.astype(jnp.float32)[:, None], (N, 8))

    x = _linear(x, params["pre_node"]["W"], params["pre_node"]["b"], "silu",
                _NODE_BLK)
    ea = _linear(edge_attr, params["pre_edge"]["W"], params["pre_edge"]["b"],
                 "silu", _EDGE_BLK)

    for name in ("layer0", "layer1"):
        lp = params[name]
        cp = lp["conv"]
        se = _linear(ea, cp["lin_e_W"], jnp.zeros((HID,), jnp.float32), None,
                     _EDGE_BLK)
        for _ in range(2):
            xl = _linear(x, cp["lin_l"]["W"], cp["lin_l"]["b"], None, _NODE_BLK)
            xr = _linear(x, cp["lin_r"]["W"], cp["lin_r"]["b"], None, _NODE_BLK)
            n0, n1, d0, d1 = _edge_phase(xl, xr, se, src, dst, cp["att"])
            x = _gru_node(n0, n1, d0, d1, cp["bias"], x, lp["gru"])
        x = _dgn_node(x, lp["norm"])

    out = _pool(x, batchf)
    gp = params["gconv"]
    xl_g = _linear(x, gp["lin_l"]["W"], gp["lin_l"]["b"], None, _NODE_BLK)
    for _ in range(2):
        xr_g = _linear(out, gp["lin_r"]["W"], gp["lin_r"]["b"], None, B)
        num, den = _ggat(xl_g, batchf, xr_g, gp["att"])
        out = _gru_graph(num, den, gp["bias"], out, params["ggru"])
    y = _dgn_graph(out, params["gnorm"])
    y = _linear(y, params["post0"]["W"], params["post0"]["b"], "silu", B)
    y = _linear(y, params["post1"]["W"], params["post1"]["b"], "silu", B)
    return _linear(y, params["out"]["W"], params["out"]["b"], None, B)


# speedup vs baseline: 6.6218x; 2.6412x over previous
import functools

import jax
import jax.numpy as jnp
from jax import lax
from jax.experimental import pallas as pl
from jax.experimental.pallas import tpu as pltpu
from jax.experimental.pallas import tpu_sc as plsc

N = 10000
E = 320000
HID = 64
B = 64
GROUPS = 10
LAMDA = 0.01
EPS = 1e-5
NEG = -1e30

_NODE_BLK = 400
_EDGE_BLK = 2000


def _silu(v):
    return v * jax.nn.sigmoid(v)


def _elu(v):
    return jnp.where(v > 0, v, jnp.exp(jnp.minimum(v, 0.0)) - 1.0)


def _pad8(v):
    return jnp.broadcast_to(v[None, :], (8, v.shape[0]))


def _lin_body(act, x_ref, wt_ref, b_ref, o_ref):
    v = jnp.dot(x_ref[...], wt_ref[...], preferred_element_type=jnp.float32)
    v = v + b_ref[0:1, :]
    if act == "silu":
        v = _silu(v)
    o_ref[...] = v


def _linear(x, W, b, act, blk):
    rows, din = x.shape
    h = W.shape[0]
    grid = rows // blk
    return pl.pallas_call(
        functools.partial(_lin_body, act),
        grid=(grid,),
        in_specs=[
            pl.BlockSpec((blk, din), lambda i: (i, 0)),
            pl.BlockSpec((din, h), lambda i: (0, 0)),
            pl.BlockSpec((8, h), lambda i: (0, 0)),
        ],
        out_specs=pl.BlockSpec((blk, h), lambda i: (i, 0)),
        out_shape=jax.ShapeDtypeStruct((rows, h), jnp.float32),
    )(x, W.T, _pad8(b))


def _gru_math(g, hprev, wr, wz, wn, ur, uz, un, br, bz, bn, cr, cz, cn):
    ir = jnp.dot(g, wr, preferred_element_type=jnp.float32) + br[0:1, :]
    iz = jnp.dot(g, wz, preferred_element_type=jnp.float32) + bz[0:1, :]
    inn = jnp.dot(g, wn, preferred_element_type=jnp.float32) + bn[0:1, :]
    hr = jnp.dot(hprev, ur, preferred_element_type=jnp.float32) + cr[0:1, :]
    hz = jnp.dot(hprev, uz, preferred_element_type=jnp.float32) + cz[0:1, :]
    hn = jnp.dot(hprev, un, preferred_element_type=jnp.float32) + cn[0:1, :]
    r = jax.nn.sigmoid(ir + hr)
    z = jax.nn.sigmoid(iz + hz)
    n = jnp.tanh(inn + r * hn)
    return jnp.maximum((1.0 - z) * n + z * hprev, 0.0)


def _gru_node_body(num0_ref, num1_ref, den0_ref, den1_ref, bias_ref, x_ref,
                   wr, wz, wn, ur, uz, un, br, bz, bn, cr, cz, cn, o_ref):
    num = num0_ref[...] + num1_ref[...]
    den = den0_ref[...][:, 0:1] + den1_ref[...][:, 0:1]
    g = _elu(num / (den + 1e-16) + bias_ref[0:1, :])
    o_ref[...] = _gru_math(g, x_ref[...], wr[...], wz[...], wn[...],
                           ur[...], uz[...], un[...], br[...], bz[...],
                           bn[...], cr[...], cz[...], cn[...])


def _split_gru(p):
    Wih, Whh = p["W_ih"], p["W_hh"]
    bih, bhh = p["b_ih"], p["b_hh"]
    outs = []
    for i in range(3):
        outs.append(Wih[i * HID:(i + 1) * HID].T)
    for i in range(3):
        outs.append(Whh[i * HID:(i + 1) * HID].T)
    for i in range(3):
        outs.append(_pad8(bih[i * HID:(i + 1) * HID]))
    for i in range(3):
        outs.append(_pad8(bhh[i * HID:(i + 1) * HID]))
    return outs


def _gru_node(num0, num1, den0, den1, bias, x, gp):
    blk = _NODE_BLK
    grid = N // blk
    wmats = _split_gru(gp)
    full = lambda s: pl.BlockSpec(s, lambda i: (0, 0))
    rowspec = lambda w: pl.BlockSpec((blk, w), lambda i: (i, 0))
    return pl.pallas_call(
        _gru_node_body,
        grid=(grid,),
        in_specs=[rowspec(HID), rowspec(HID), rowspec(16), rowspec(16),
                  full((8, HID)), rowspec(HID)]
                 + [full((HID, HID))] * 6 + [full((8, HID))] * 6,
        out_specs=rowspec(HID),
        out_shape=jax.ShapeDtypeStruct((N, HID), jnp.float32),
    )(num0, num1, den0, den1, _pad8(bias), x, *wmats)


def _gru_graph_body(num_ref, den_ref, bias_ref, h_ref,
                    wr, wz, wn, ur, uz, un, br, bz, bn, cr, cz, cn, o_ref):
    g = _elu(num_ref[...] / (den_ref[...][:, 0:1] + 1e-16) + bias_ref[0:1, :])
    o_ref[...] = _gru_math(g, h_ref[...], wr[...], wz[...], wn[...],
                           ur[...], uz[...], un[...], br[...], bz[...],
                           bn[...], cr[...], cz[...], cn[...])


def _gru_graph(num, den, bias, h, gp):
    wmats = _split_gru(gp)
    return pl.pallas_call(
        _gru_graph_body,
        out_shape=jax.ShapeDtypeStruct((B, HID), jnp.float32),
    )(num, den, _pad8(bias), h, *wmats)


def _softmax_s(xb, wnt, bn16):
    logits = jnp.dot(xb, wnt, preferred_element_type=jnp.float32) + bn16[0:1, :]
    m = jnp.max(logits, axis=1, keepdims=True)
    s = jnp.exp(logits - m)
    return s / jnp.sum(s, axis=1, keepdims=True)


def _dgn_stats_body(x_ref, wnt_ref, bn_ref, m_ref, q_ref):
    i = pl.program_id(0)
    xb = x_ref[...]
    s = _softmax_s(xb, wnt_ref[...], bn_ref[...])
    mp = lax.dot_general(s, xb, (((0,), (0,)), ((), ())),
                         preferred_element_type=jnp.float32)
    qp = lax.dot_general(s * s, xb * xb, (((0,), (0,)), ((), ())),
                         preferred_element_type=jnp.float32)

    @pl.when(i == 0)
    def _():
        m_ref[...] = jnp.zeros_like(m_ref)
        q_ref[...] = jnp.zeros_like(q_ref)

    m_ref[...] += mp
    q_ref[...] += qp


def _dgn_apply_body(nrows, x_ref, wnt_ref, bn_ref, m_ref, q_ref,
                    bw_ref, bb_ref, o_ref):
    xb = x_ref[...]
    s = _softmax_s(xb, wnt_ref[...], bn_ref[...])
    mean = m_ref[...] * (1.0 / nrows)
    var = q_ref[...] * (1.0 / nrows) - mean * mean
    rstd = lax.rsqrt(var + EPS)
    wsc = bw_ref[...] * rstd
    a = jnp.dot(s, wsc, preferred_element_type=jnp.float32)
    c = jnp.sum(mean * wsc - bb_ref[...], axis=0, keepdims=True)
    o_ref[...] = xb + LAMDA * (xb * a - c)


def _dgn_prep(p):
    W, b = p["lin"]["W"], p["lin"]["b"]
    wnt = jnp.zeros((HID, 16), jnp.float32).at[:, :GROUPS].set(W.T)
    bn16 = jnp.full((16,), NEG, jnp.float32).at[:GROUPS].set(b)
    bw = jnp.zeros((16, HID), jnp.float32).at[:GROUPS].set(
        p["bn_w"].reshape(GROUPS, HID))
    bb = jnp.zeros((16, HID), jnp.float32).at[:GROUPS].set(
        p["bn_b"].reshape(GROUPS, HID))
    return wnt, _pad8(bn16), bw, bb


def _dgn_node(x, p):
    wnt, bn16, bw, bb = _dgn_prep(p)
    blk = _NODE_BLK
    grid = N // blk
    full = lambda s: pl.BlockSpec(s, lambda i: (0, 0))
    m, q = pl.pallas_call(
        _dgn_stats_body,
        grid=(grid,),
        in_specs=[pl.BlockSpec((blk, HID), lambda i: (i, 0)),
                  full((HID, 16)), full((8, 16))],
        out_specs=[full((16, HID)), full((16, HID))],
        out_shape=[jax.ShapeDtypeStruct((16, HID), jnp.float32)] * 2,
    )(x, wnt, bn16)
    return pl.pallas_call(
        functools.partial(_dgn_apply_body, float(N)),
        grid=(grid,),
        in_specs=[pl.BlockSpec((blk, HID), lambda i: (i, 0)),
                  full((HID, 16)), full((8, 16)), full((16, HID)),
                  full((16, HID)), full((16, HID)), full((16, HID))],
        out_specs=pl.BlockSpec((blk, HID), lambda i: (i, 0)),
        out_shape=jax.ShapeDtypeStruct((N, HID), jnp.float32),
    )(x, wnt, bn16, m, q, bw, bb)


def _dgn_graph_body(x_ref, wnt_ref, bn_ref, bw_ref, bb_ref, o_ref):
    xb = x_ref[...]
    s = _softmax_s(xb, wnt_ref[...], bn_ref[...])
    mp = lax.dot_general(s, xb, (((0,), (0,)), ((), ())),
                         preferred_element_type=jnp.float32)
    qp = lax.dot_general(s * s, xb * xb, (((0,), (0,)), ((), ())),
                         preferred_element_type=jnp.float32)
    mean = mp * (1.0 / B)
    var = qp * (1.0 / B) - mean * mean
    rstd = lax.rsqrt(var + EPS)
    wsc = bw_ref[...] * rstd
    a = jnp.dot(s, wsc, preferred_element_type=jnp.float32)
    c = jnp.sum(mean * wsc - bb_ref[...], axis=0, keepdims=True)
    o_ref[...] = xb + LAMDA * (xb * a - c)


def _dgn_graph(x, p):
    wnt, bn16, bw, bb = _dgn_prep(p)
    return pl.pallas_call(
        _dgn_graph_body,
        out_shape=jax.ShapeDtypeStruct((B, HID), jnp.float32),
    )(x, wnt, bn16, bw, bb)


def _pool_body(nblocks, bf_ref, x_ref, o_ref):
    i = pl.program_id(0)
    iota = lax.broadcasted_iota(jnp.int32, (1, B), 1).astype(jnp.float32)
    onehot = (bf_ref[...][:, 0:1] == iota).astype(jnp.float32)
    part = lax.dot_general(onehot, x_ref[...], (((0,), (0,)), ((), ())),
                           preferred_element_type=jnp.float32)

    @pl.when(i == 0)
    def _():
        o_ref[...] = jnp.zeros_like(o_ref)

    o_ref[...] += part

    @pl.when(i == nblocks - 1)
    def _():
        o_ref[...] = jnp.maximum(o_ref[...], 0.0)


def _pool(x, batchf):
    blk = _NODE_BLK
    grid = N // blk
    return pl.pallas_call(
        functools.partial(_pool_body, grid),
        grid=(grid,),
        in_specs=[pl.BlockSpec((blk, 8), lambda i: (i, 0)),
                  pl.BlockSpec((blk, HID), lambda i: (i, 0))],
        out_specs=pl.BlockSpec((B, HID), lambda i: (0, 0)),
        out_shape=jax.ShapeDtypeStruct((B, HID), jnp.float32),
    )(batchf, x)


def _ggat_alpha(xlb, bf, xr, att):
    iota = lax.broadcasted_iota(jnp.int32, (1, B), 1).astype(jnp.float32)
    onehot = (bf[:, 0:1] == iota).astype(jnp.float32)
    e = xlb + jnp.dot(onehot, xr, preferred_element_type=jnp.float32)
    e = jnp.where(e > 0, e, 0.01 * e)
    alpha = jnp.dot(e, att, preferred_element_type=jnp.float32)
    return onehot, alpha


def _ggat1_body(nblocks, xl_ref, bf_ref, xr_ref, att_ref, amax_ref):
    i = pl.program_id(0)
    onehot, alpha = _ggat_alpha(xl_ref[...], bf_ref[...], xr_ref[...],
                                att_ref[...][:, 0:1])
    masked = jnp.where(onehot > 0, alpha, NEG)
    pmax = jnp.max(masked, axis=0, keepdims=True)

    @pl.when(i == 0)
    def _():
        amax_ref[...] = jnp.full_like(amax_ref, NEG)

    amax_ref[...] = jnp.maximum(amax_ref[...], jnp.broadcast_to(pmax, (8, B)))


def _ggat2_body(xl_ref, bf_ref, xr_ref, att_ref, amax_ref, num_ref, den_ref):
    i = pl.program_id(0)
    onehot, alpha = _ggat_alpha(xl_ref[...], bf_ref[...], xr_ref[...],
                                att_ref[...][:, 0:1])
    am = amax_ref[...][0:1, :]
    am = jnp.where(am < -1e29, 0.0, am)
    amrow = jnp.sum(onehot * am, axis=1, keepdims=True)
    ex = jnp.exp(alpha - amrow)
    np_ = lax.dot_general(onehot, ex * xl_ref[...], (((0,), (0,)), ((), ())),
                          preferred_element_type=jnp.float32)
    dp = lax.dot_general(onehot, jnp.broadcast_to(ex, ex.shape[:1] + (8,)),
                         (((0,), (0,)), ((), ())),
                         preferred_element_type=jnp.float32)

    @pl.when(i == 0)
    def _():
        num_ref[...] = jnp.zeros_like(num_ref)
        den_ref[...] = jnp.zeros_like(den_ref)

    num_ref[...] += np_
    den_ref[...] += dp


def _ggat(xl, batchf, xr, att):
    blk = _NODE_BLK
    grid = N // blk
    full = lambda s: pl.BlockSpec(s, lambda i: (0, 0))
    att2 = jnp.broadcast_to(att[:, None], (HID, 8))
    amax = pl.pallas_call(
        functools.partial(_ggat1_body, grid),
        grid=(grid,),
        in_specs=[pl.BlockSpec((blk, HID), lambda i: (i, 0)),
                  pl.BlockSpec((blk, 8), lambda i: (i, 0)),
                  full((B, HID)), full((HID, 8))],
        out_specs=full((8, B)),
        out_shape=jax.ShapeDtypeStruct((8, B), jnp.float32),
    )(xl, batchf, xr, att2)
    num, den = pl.pallas_call(
        _ggat2_body,
        grid=(grid,),
        in_specs=[pl.BlockSpec((blk, HID), lambda i: (i, 0)),
                  pl.BlockSpec((blk, 8), lambda i: (i, 0)),
                  full((B, HID)), full((HID, 8)), full((8, B))],
        out_specs=[full((B, HID)), full((B, 8))],
        out_shape=[jax.ShapeDtypeStruct((B, HID), jnp.float32),
                   jax.ShapeDtypeStruct((B, 8), jnp.float32)],
    )(xl, batchf, xr, att2, amax)
    return num, den


_NC, _NS, _L = 2, 16, 16
_NW = _NC * _NS
_EPT = E // _NW
_K = 80
_NCHUNK = _EPT // _K
_CPY = 640
_CSTEP = 624

_MESH = plsc.VectorSubcoreMesh(core_axis_name="c", subcore_axis_name="s")


def _lane_iota():
    return lax.broadcasted_iota(jnp.int32, (_L,), 0)


def _shuf(v, idx):
    dnums = lax.GatherDimensionNumbers(
        offset_dims=(), collapsed_slice_dims=(0,), start_index_map=(0,))
    return lax.gather(v, idx[:, None], dnums, slice_sizes=(1,),
                      mode=lax.GatherScatterMode.PROMISE_IN_BOUNDS)


def _lane_sum(v):
    lane = _lane_iota()
    for s in (8, 4, 2, 1):
        v = v + _shuf(v, lane ^ s)
    return v


def _lane_max(v):
    lane = _lane_iota()
    for s in (8, 4, 2, 1):
        v = jnp.maximum(v, _shuf(v, lane ^ s))
    return v


def _sc_alpha(xlr, se, src, dst, att):
    @functools.partial(
        pl.kernel,
        out_type=[jax.ShapeDtypeStruct((E,), jnp.float32),
                  jax.ShapeDtypeStruct((_NW, 16), jnp.float32)],
        mesh=_MESH,
        scratch_types=[
            pltpu.VMEM((_K,), jnp.int32),
            pltpu.VMEM((_K,), jnp.int32),
            pltpu.VMEM((_K, 2 * HID), jnp.float32),
            pltpu.VMEM((_K, 2 * HID), jnp.float32),
            pltpu.VMEM((_K, HID), jnp.float32),
            pltpu.VMEM((_K,), jnp.float32),
            pltpu.VMEM((HID,), jnp.float32),
            pltpu.VMEM((16,), jnp.float32),
            pltpu.SemaphoreType.DMA,
            pltpu.SemaphoreType.DMA,
        ],
    )
    def body(xlr_h, se_h, src_h, dst_h, att_h, alpha_h, tmax_h,
             srcv, dstv, gsrc, gdst, sev, alv, attv, mxv, sem1, sem2):
        cid = lax.axis_index("c")
        sid = lax.axis_index("s")
        wid = sid * _NC + cid
        pltpu.sync_copy(att_h, attv)

        def chunk(ci, mx):
            base = wid * _EPT + ci * _K
            pltpu.sync_copy(src_h.at[pl.ds(base, _K)], srcv)
            pltpu.sync_copy(dst_h.at[pl.ds(base, _K)], dstv)
            cp1 = pltpu.async_copy(xlr_h.at[srcv], gsrc, sem1)
            cp2 = pltpu.async_copy(xlr_h.at[dstv], gdst, sem2)
            pltpu.sync_copy(se_h.at[pl.ds(base, _K)], sev)
            cp1.wait()
            cp2.wait()
            for g in range(_K // _L):
                av = jnp.zeros((_L,), jnp.float32)
                for j in range(_L):
                    i = g * _L + j
                    acc = jnp.zeros((_L,), jnp.float32)
                    for f in range(HID // _L):
                        v = (gsrc[i, pl.ds(f * _L, _L)]
                             + gdst[i, pl.ds(HID + f * _L, _L)]
                             + sev[i, pl.ds(f * _L, _L)])
                        v = jnp.maximum(v, 0.0) + 0.01 * jnp.minimum(v, 0.0)
                        acc = acc + v * attv[pl.ds(f * _L, _L)]
                    a = _lane_sum(acc)
                    av = jnp.where(_lane_iota() == j, a, av)
                alv[pl.ds(g * _L, _L)] = av
                mx = jnp.maximum(mx, av)
            pltpu.sync_copy(alv, alpha_h.at[pl.ds(base, _K)])
            return mx

        mx = lax.fori_loop(0, _NCHUNK, chunk,
                           jnp.full((_L,), NEG, jnp.float32))
        mxv[...] = mx
        pltpu.sync_copy(mxv, tmax_h.at[wid])

    return body(xlr, se, src, dst, att)


def _sc_scatter(xlr, src, dst, alpha, tmax, zn):
    @functools.partial(
        pl.kernel,
        out_type=jax.ShapeDtypeStruct((2 * N, 2 * HID), jnp.float32),
        mesh=_MESH,
        scratch_types=[
            pltpu.VMEM((_K,), jnp.int32),
            pltpu.VMEM((_K,), jnp.int32),
            pltpu.VMEM((_K,), jnp.float32),
            pltpu.VMEM((_K, 2 * HID), jnp.float32),
            pltpu.VMEM((_K, 2 * HID), jnp.float32),
            pltpu.VMEM((_NW, 16), jnp.float32),
            pltpu.VMEM((_K, 2 * HID), jnp.float32),
            pltpu.VMEM_SHARED((N, 2 * HID), jnp.float32),
            pltpu.SemaphoreType.DMA,
        ],
    )
    def body(xlr_h, src_h, dst_h, alpha_h, tmax_h, zn_h, acc_h,
             srcv, dstv, alv, xlv, scn, tmaxv, bn, acc_s, sem1):
        cid = lax.axis_index("c")
        sid = lax.axis_index("s")
        wid = sid * _NC + cid

        pltpu.sync_copy(zn_h, bn)
        for k in range(_CPY // _K):
            pltpu.sync_copy(bn, acc_s.at[pl.ds(sid * _CSTEP + k * _K, _K)])

        zero = jnp.zeros((_L,), jnp.float32)
        for i in range(_K):
            for f in range(HID + _L, 2 * HID, _L):
                scn[i, pl.ds(f, _L)] = zero

        pltpu.sync_copy(tmax_h, tmaxv)
        mm = tmaxv[0, :]
        for r in range(1, _NW):
            mm = jnp.maximum(mm, tmaxv[r, :])
        gmax = _lane_max(mm)

        plsc.subcore_barrier()

        def chunk(ci, carry):
            base = wid * _EPT + ci * _K
            pltpu.sync_copy(src_h.at[pl.ds(base, _K)], srcv)
            pltpu.sync_copy(dst_h.at[pl.ds(base, _K)], dstv)
            cp1 = pltpu.async_copy(xlr_h.at[srcv], xlv, sem1)
            pltpu.sync_copy(alpha_h.at[pl.ds(base, _K)], alv)
            cp1.wait()
            lane = _lane_iota()
            for g in range(_K // _L):
                ex16 = jnp.exp(alv[pl.ds(g * _L, _L)] - gmax)
                for j in range(_L):
                    i = g * _L + j
                    exb = _shuf(ex16, jnp.full((_L,), j, jnp.int32))
                    for f in range(HID // _L):
                        scn[i, pl.ds(f * _L, _L)] = (
                            xlv[i, pl.ds(f * _L, _L)] * exb)
                    scn[i, pl.ds(HID, _L)] = jnp.where(lane == 0, exb, 0.0)
            pltpu.sync_copy(scn, acc_s.at[dstv], add=True)
            return carry

        lax.fori_loop(0, _NCHUNK, chunk, jnp.int32(0))

        plsc.subcore_barrier()

        for k in range(_CPY // _K):
            off = sid * _CSTEP + k * _K
            pltpu.sync_copy(acc_s.at[pl.ds(off, _K)], bn)
            pltpu.sync_copy(bn, acc_h.at[pl.ds(cid * N + off, _K)])

    return body(xlr, src, dst, alpha, tmax, zn)


def _edge_phase(xlr, se, src, dst, att, zn):
    alpha, tmax = _sc_alpha(xlr, se, src, dst, att)
    acc = _sc_scatter(xlr, src, dst, alpha, tmax, zn)
    return (acc[:N, :HID], acc[N:, :HID],
            acc[:N, HID:HID + 16], acc[N:, HID:HID + 16])


def kernel(x, edge_index, edge_attr, batch, params):
    src, dst = edge_index[0], edge_index[1]
    batchf = jnp.broadcast_to(batch.astype(jnp.float32)[:, None], (N, 8))
    zn = jnp.zeros((_K, 2 * HID), jnp.float32)

    x = _linear(x, params["pre_node"]["W"], params["pre_node"]["b"], "silu",
                _NODE_BLK)
    ea = _linear(edge_attr, params["pre_edge"]["W"], params["pre_edge"]["b"],
                 "silu", _EDGE_BLK)

    for name in ("layer0", "layer1"):
        lp = params[name]
        cp = lp["conv"]
        se = _linear(ea, cp["lin_e_W"], jnp.zeros((HID,), jnp.float32), None,
                     _EDGE_BLK)
        W_lr = jnp.concatenate([cp["lin_l"]["W"], cp["lin_r"]["W"]], axis=0)
        b_lr = jnp.concatenate([cp["lin_l"]["b"], cp["lin_r"]["b"]])
        for _ in range(2):
            xlr = _linear(x, W_lr, b_lr, None, _NODE_BLK)
            n0, n1, d0, d1 = _edge_phase(xlr, se, src, dst, cp["att"], zn)
            x = _gru_node(n0, n1, d0, d1, cp["bias"], x, lp["gru"])
        x = _dgn_node(x, lp["norm"])

    out = _pool(x, batchf)
    gp = params["gconv"]
    xl_g = _linear(x, gp["lin_l"]["W"], gp["lin_l"]["b"], None, _NODE_BLK)
    for _ in range(2):
        xr_g = _linear(out, gp["lin_r"]["W"], gp["lin_r"]["b"], None, B)
        num, den = _ggat(xl_g, batchf, xr_g, gp["att"])
        out = _gru_graph(num, den, gp["bias"], out, params["ggru"])
    y = _dgn_graph(out, params["gnorm"])
    y = _linear(y, params["post0"]["W"], params["post0"]["b"], "silu", B)
    y = _linear(y, params["post1"]["W"], params["post1"]["b"], "silu", B)
    return _linear(y, params["out"]["W"], params["out"]["b"], None, B)


# speedup vs baseline: 6.6415x; 1.0030x over previous
import functools

import jax
import jax.numpy as jnp
from jax import lax
from jax.experimental import pallas as pl
from jax.experimental.pallas import tpu as pltpu
from jax.experimental.pallas import tpu_sc as plsc

N = 10000
E = 320000
HID = 64
B = 64
GROUPS = 10
LAMDA = 0.01
EPS = 1e-5
NEG = -1e30

_NODE_BLK = 400
_EDGE_BLK = 2000


def _silu(v):
    return v * jax.nn.sigmoid(v)


def _elu(v):
    return jnp.where(v > 0, v, jnp.exp(jnp.minimum(v, 0.0)) - 1.0)


def _pad8(v):
    return jnp.broadcast_to(v[None, :], (8, v.shape[0]))


def _lin_body(act, x_ref, wt_ref, b_ref, o_ref):
    v = jnp.dot(x_ref[...], wt_ref[...], preferred_element_type=jnp.float32)
    v = v + b_ref[0:1, :]
    if act == "silu":
        v = _silu(v)
    o_ref[...] = v


def _linear(x, W, b, act, blk):
    rows, din = x.shape
    h = W.shape[0]
    grid = rows // blk
    return pl.pallas_call(
        functools.partial(_lin_body, act),
        grid=(grid,),
        in_specs=[
            pl.BlockSpec((blk, din), lambda i: (i, 0)),
            pl.BlockSpec((din, h), lambda i: (0, 0)),
            pl.BlockSpec((8, h), lambda i: (0, 0)),
        ],
        out_specs=pl.BlockSpec((blk, h), lambda i: (i, 0)),
        out_shape=jax.ShapeDtypeStruct((rows, h), jnp.float32),
    )(x, W.T, _pad8(b))


def _gru_math(g, hprev, wr, wz, wn, ur, uz, un, br, bz, bn, cr, cz, cn):
    ir = jnp.dot(g, wr, preferred_element_type=jnp.float32) + br[0:1, :]
    iz = jnp.dot(g, wz, preferred_element_type=jnp.float32) + bz[0:1, :]
    inn = jnp.dot(g, wn, preferred_element_type=jnp.float32) + bn[0:1, :]
    hr = jnp.dot(hprev, ur, preferred_element_type=jnp.float32) + cr[0:1, :]
    hz = jnp.dot(hprev, uz, preferred_element_type=jnp.float32) + cz[0:1, :]
    hn = jnp.dot(hprev, un, preferred_element_type=jnp.float32) + cn[0:1, :]
    r = jax.nn.sigmoid(ir + hr)
    z = jax.nn.sigmoid(iz + hz)
    n = jnp.tanh(inn + r * hn)
    return jnp.maximum((1.0 - z) * n + z * hprev, 0.0)


def _gru_node_body(num0_ref, num1_ref, den0_ref, den1_ref, bias_ref, x_ref,
                   wr, wz, wn, ur, uz, un, br, bz, bn, cr, cz, cn, o_ref):
    num = num0_ref[...] + num1_ref[...]
    den = den0_ref[...][:, 0:1] + den1_ref[...][:, 0:1]
    g = _elu(num / (den + 1e-16) + bias_ref[0:1, :])
    o_ref[...] = _gru_math(g, x_ref[...], wr[...], wz[...], wn[...],
                           ur[...], uz[...], un[...], br[...], bz[...],
                           bn[...], cr[...], cz[...], cn[...])


def _split_gru(p):
    Wih, Whh = p["W_ih"], p["W_hh"]
    bih, bhh = p["b_ih"], p["b_hh"]
    outs = []
    for i in range(3):
        outs.append(Wih[i * HID:(i + 1) * HID].T)
    for i in range(3):
        outs.append(Whh[i * HID:(i + 1) * HID].T)
    for i in range(3):
        outs.append(_pad8(bih[i * HID:(i + 1) * HID]))
    for i in range(3):
        outs.append(_pad8(bhh[i * HID:(i + 1) * HID]))
    return outs


def _gru_node(num0, num1, den0, den1, bias, x, gp):
    blk = _NODE_BLK
    grid = N // blk
    wmats = _split_gru(gp)
    full = lambda s: pl.BlockSpec(s, lambda i: (0, 0))
    rowspec = lambda w: pl.BlockSpec((blk, w), lambda i: (i, 0))
    return pl.pallas_call(
        _gru_node_body,
        grid=(grid,),
        in_specs=[rowspec(HID), rowspec(HID), rowspec(16), rowspec(16),
                  full((8, HID)), rowspec(HID)]
                 + [full((HID, HID))] * 6 + [full((8, HID))] * 6,
        out_specs=rowspec(HID),
        out_shape=jax.ShapeDtypeStruct((N, HID), jnp.float32),
    )(num0, num1, den0, den1, _pad8(bias), x, *wmats)


def _gru_graph_body(num_ref, den_ref, bias_ref, h_ref,
                    wr, wz, wn, ur, uz, un, br, bz, bn, cr, cz, cn, o_ref):
    g = _elu(num_ref[...] / (den_ref[...][:, 0:1] + 1e-16) + bias_ref[0:1, :])
    o_ref[...] = _gru_math(g, h_ref[...], wr[...], wz[...], wn[...],
                           ur[...], uz[...], un[...], br[...], bz[...],
                           bn[...], cr[...], cz[...], cn[...])


def _gru_graph(num, den, bias, h, gp):
    wmats = _split_gru(gp)
    return pl.pallas_call(
        _gru_graph_body,
        out_shape=jax.ShapeDtypeStruct((B, HID), jnp.float32),
    )(num, den, _pad8(bias), h, *wmats)


def _softmax_s(xb, wnt, bn16):
    logits = jnp.dot(xb, wnt, preferred_element_type=jnp.float32) + bn16[0:1, :]
    m = jnp.max(logits, axis=1, keepdims=True)
    s = jnp.exp(logits - m)
    return s / jnp.sum(s, axis=1, keepdims=True)


def _dgn_stats_body(x_ref, wnt_ref, bn_ref, m_ref, q_ref):
    i = pl.program_id(0)
    xb = x_ref[...]
    s = _softmax_s(xb, wnt_ref[...], bn_ref[...])
    mp = lax.dot_general(s, xb, (((0,), (0,)), ((), ())),
                         preferred_element_type=jnp.float32)
    qp = lax.dot_general(s * s, xb * xb, (((0,), (0,)), ((), ())),
                         preferred_element_type=jnp.float32)

    @pl.when(i == 0)
    def _():
        m_ref[...] = jnp.zeros_like(m_ref)
        q_ref[...] = jnp.zeros_like(q_ref)

    m_ref[...] += mp
    q_ref[...] += qp


def _dgn_apply_body(nrows, x_ref, wnt_ref, bn_ref, m_ref, q_ref,
                    bw_ref, bb_ref, o_ref):
    xb = x_ref[...]
    s = _softmax_s(xb, wnt_ref[...], bn_ref[...])
    mean = m_ref[...] * (1.0 / nrows)
    var = q_ref[...] * (1.0 / nrows) - mean * mean
    rstd = lax.rsqrt(var + EPS)
    wsc = bw_ref[...] * rstd
    a = jnp.dot(s, wsc, preferred_element_type=jnp.float32)
    c = jnp.sum(mean * wsc - bb_ref[...], axis=0, keepdims=True)
    o_ref[...] = xb + LAMDA * (xb * a - c)


def _dgn_prep(p):
    W, b = p["lin"]["W"], p["lin"]["b"]
    wnt = jnp.zeros((HID, 16), jnp.float32).at[:, :GROUPS].set(W.T)
    bn16 = jnp.full((16,), NEG, jnp.float32).at[:GROUPS].set(b)
    bw = jnp.zeros((16, HID), jnp.float32).at[:GROUPS].set(
        p["bn_w"].reshape(GROUPS, HID))
    bb = jnp.zeros((16, HID), jnp.float32).at[:GROUPS].set(
        p["bn_b"].reshape(GROUPS, HID))
    return wnt, _pad8(bn16), bw, bb


def _dgn_node(x, p):
    wnt, bn16, bw, bb = _dgn_prep(p)
    blk = _NODE_BLK
    grid = N // blk
    full = lambda s: pl.BlockSpec(s, lambda i: (0, 0))
    m, q = pl.pallas_call(
        _dgn_stats_body,
        grid=(grid,),
        in_specs=[pl.BlockSpec((blk, HID), lambda i: (i, 0)),
                  full((HID, 16)), full((8, 16))],
        out_specs=[full((16, HID)), full((16, HID))],
        out_shape=[jax.ShapeDtypeStruct((16, HID), jnp.float32)] * 2,
    )(x, wnt, bn16)
    return pl.pallas_call(
        functools.partial(_dgn_apply_body, float(N)),
        grid=(grid,),
        in_specs=[pl.BlockSpec((blk, HID), lambda i: (i, 0)),
                  full((HID, 16)), full((8, 16)), full((16, HID)),
                  full((16, HID)), full((16, HID)), full((16, HID))],
        out_specs=pl.BlockSpec((blk, HID), lambda i: (i, 0)),
        out_shape=jax.ShapeDtypeStruct((N, HID), jnp.float32),
    )(x, wnt, bn16, m, q, bw, bb)


def _dgn_graph_body(x_ref, wnt_ref, bn_ref, bw_ref, bb_ref, o_ref):
    xb = x_ref[...]
    s = _softmax_s(xb, wnt_ref[...], bn_ref[...])
    mp = lax.dot_general(s, xb, (((0,), (0,)), ((), ())),
                         preferred_element_type=jnp.float32)
    qp = lax.dot_general(s * s, xb * xb, (((0,), (0,)), ((), ())),
                         preferred_element_type=jnp.float32)
    mean = mp * (1.0 / B)
    var = qp * (1.0 / B) - mean * mean
    rstd = lax.rsqrt(var + EPS)
    wsc = bw_ref[...] * rstd
    a = jnp.dot(s, wsc, preferred_element_type=jnp.float32)
    c = jnp.sum(mean * wsc - bb_ref[...], axis=0, keepdims=True)
    o_ref[...] = xb + LAMDA * (xb * a - c)


def _dgn_graph(x, p):
    wnt, bn16, bw, bb = _dgn_prep(p)
    return pl.pallas_call(
        _dgn_graph_body,
        out_shape=jax.ShapeDtypeStruct((B, HID), jnp.float32),
    )(x, wnt, bn16, bw, bb)


def _pool_body(nblocks, bf_ref, x_ref, o_ref):
    i = pl.program_id(0)
    iota = lax.broadcasted_iota(jnp.int32, (1, B), 1).astype(jnp.float32)
    onehot = (bf_ref[...][:, 0:1] == iota).astype(jnp.float32)
    part = lax.dot_general(onehot, x_ref[...], (((0,), (0,)), ((), ())),
                           preferred_element_type=jnp.float32)

    @pl.when(i == 0)
    def _():
        o_ref[...] = jnp.zeros_like(o_ref)

    o_ref[...] += part

    @pl.when(i == nblocks - 1)
    def _():
        o_ref[...] = jnp.maximum(o_ref[...], 0.0)


def _pool(x, batchf):
    blk = _NODE_BLK
    grid = N // blk
    return pl.pallas_call(
        functools.partial(_pool_body, grid),
        grid=(grid,),
        in_specs=[pl.BlockSpec((blk, 8), lambda i: (i, 0)),
                  pl.BlockSpec((blk, HID), lambda i: (i, 0))],
        out_specs=pl.BlockSpec((B, HID), lambda i: (0, 0)),
        out_shape=jax.ShapeDtypeStruct((B, HID), jnp.float32),
    )(batchf, x)


def _ggat_alpha(xlb, bf, xr, att):
    iota = lax.broadcasted_iota(jnp.int32, (1, B), 1).astype(jnp.float32)
    onehot = (bf[:, 0:1] == iota).astype(jnp.float32)
    e = xlb + jnp.dot(onehot, xr, preferred_element_type=jnp.float32)
    e = jnp.where(e > 0, e, 0.01 * e)
    alpha = jnp.dot(e, att, preferred_element_type=jnp.float32)
    return onehot, alpha


def _ggat1_body(nblocks, xl_ref, bf_ref, xr_ref, att_ref, amax_ref):
    i = pl.program_id(0)
    onehot, alpha = _ggat_alpha(xl_ref[...], bf_ref[...], xr_ref[...],
                                att_ref[...][:, 0:1])
    masked = jnp.where(onehot > 0, alpha, NEG)
    pmax = jnp.max(masked, axis=0, keepdims=True)

    @pl.when(i == 0)
    def _():
        amax_ref[...] = jnp.full_like(amax_ref, NEG)

    amax_ref[...] = jnp.maximum(amax_ref[...], jnp.broadcast_to(pmax, (8, B)))


def _ggat2_body(xl_ref, bf_ref, xr_ref, att_ref, amax_ref, num_ref, den_ref):
    i = pl.program_id(0)
    onehot, alpha = _ggat_alpha(xl_ref[...], bf_ref[...], xr_ref[...],
                                att_ref[...][:, 0:1])
    am = amax_ref[...][0:1, :]
    am = jnp.where(am < -1e29, 0.0, am)
    amrow = jnp.sum(onehot * am, axis=1, keepdims=True)
    ex = jnp.exp(alpha - amrow)
    np_ = lax.dot_general(onehot, ex * xl_ref[...], (((0,), (0,)), ((), ())),
                          preferred_element_type=jnp.float32)
    dp = lax.dot_general(onehot, jnp.broadcast_to(ex, ex.shape[:1] + (8,)),
                         (((0,), (0,)), ((), ())),
                         preferred_element_type=jnp.float32)

    @pl.when(i == 0)
    def _():
        num_ref[...] = jnp.zeros_like(num_ref)
        den_ref[...] = jnp.zeros_like(den_ref)

    num_ref[...] += np_
    den_ref[...] += dp


def _ggat(xl, batchf, xr, att):
    blk = _NODE_BLK
    grid = N // blk
    full = lambda s: pl.BlockSpec(s, lambda i: (0, 0))
    att2 = jnp.broadcast_to(att[:, None], (HID, 8))
    amax = pl.pallas_call(
        functools.partial(_ggat1_body, grid),
        grid=(grid,),
        in_specs=[pl.BlockSpec((blk, HID), lambda i: (i, 0)),
                  pl.BlockSpec((blk, 8), lambda i: (i, 0)),
                  full((B, HID)), full((HID, 8))],
        out_specs=full((8, B)),
        out_shape=jax.ShapeDtypeStruct((8, B), jnp.float32),
    )(xl, batchf, xr, att2)
    num, den = pl.pallas_call(
        _ggat2_body,
        grid=(grid,),
        in_specs=[pl.BlockSpec((blk, HID), lambda i: (i, 0)),
                  pl.BlockSpec((blk, 8), lambda i: (i, 0)),
                  full((B, HID)), full((HID, 8)), full((8, B))],
        out_specs=[full((B, HID)), full((B, 8))],
        out_shape=[jax.ShapeDtypeStruct((B, HID), jnp.float32),
                   jax.ShapeDtypeStruct((B, 8), jnp.float32)],
    )(xl, batchf, xr, att2, amax)
    return num, den


_NC, _NS, _L = 2, 16, 16
_NW = _NC * _NS
_EPT = E // _NW
_K = 80
_NCHUNK = _EPT // _K
_CPY = 640
_CSTEP = 624

_MESH = plsc.VectorSubcoreMesh(core_axis_name="c", subcore_axis_name="s")


def _lane_iota():
    return lax.broadcasted_iota(jnp.int32, (_L,), 0)


def _shuf(v, idx):
    dnums = lax.GatherDimensionNumbers(
        offset_dims=(), collapsed_slice_dims=(0,), start_index_map=(0,))
    return lax.gather(v, idx[:, None], dnums, slice_sizes=(1,),
                      mode=lax.GatherScatterMode.PROMISE_IN_BOUNDS)


def _lane_sum(v):
    lane = _lane_iota()
    for s in (8, 4, 2, 1):
        v = v + _shuf(v, lane ^ s)
    return v


def _lane_max(v):
    lane = _lane_iota()
    for s in (8, 4, 2, 1):
        v = jnp.maximum(v, _shuf(v, lane ^ s))
    return v


def _alpha_compute(gsrc, gdst, sev, attv, alv, mx):
    for g in range(_K // _L):
        av = jnp.zeros((_L,), jnp.float32)
        for j in range(_L):
            i = g * _L + j
            acc = jnp.zeros((_L,), jnp.float32)
            for f in range(HID // _L):
                v = (gsrc[i, pl.ds(f * _L, _L)]
                     + gdst[i, pl.ds(HID + f * _L, _L)]
                     + sev[i, pl.ds(f * _L, _L)])
                v = jnp.maximum(v, 0.0) + 0.01 * jnp.minimum(v, 0.0)
                acc = acc + v * attv[pl.ds(f * _L, _L)]
            a = _lane_sum(acc)
            av = jnp.where(_lane_iota() == j, a, av)
        alv[pl.ds(g * _L, _L)] = av
        mx = jnp.maximum(mx, av)
    return mx


def _sc_alpha(xlr, se, src, dst, att):
    @functools.partial(
        pl.kernel,
        out_type=[jax.ShapeDtypeStruct((E,), jnp.float32),
                  jax.ShapeDtypeStruct((_NW, 16), jnp.float32)],
        mesh=_MESH,
        scratch_types=[
            pltpu.VMEM((_K,), jnp.int32),
            pltpu.VMEM((_K,), jnp.int32),
            pltpu.VMEM((_K,), jnp.int32),
            pltpu.VMEM((_K,), jnp.int32),
            pltpu.VMEM((_K, 2 * HID), jnp.float32),
            pltpu.VMEM((_K, 2 * HID), jnp.float32),
            pltpu.VMEM((_K, 2 * HID), jnp.float32),
            pltpu.VMEM((_K, 2 * HID), jnp.float32),
            pltpu.VMEM((_K, HID), jnp.float32),
            pltpu.VMEM((_K, HID), jnp.float32),
            pltpu.VMEM((_K,), jnp.float32),
            pltpu.VMEM((_K,), jnp.float32),
            pltpu.VMEM((HID,), jnp.float32),
            pltpu.VMEM((16,), jnp.float32),
            pltpu.SemaphoreType.DMA,
            pltpu.SemaphoreType.DMA,
            pltpu.SemaphoreType.DMA,
            pltpu.SemaphoreType.DMA,
        ],
    )
    def body(xlr_h, se_h, src_h, dst_h, att_h, alpha_h, tmax_h,
             srcv0, srcv1, dstv0, dstv1, gsrc0, gsrc1, gdst0, gdst1,
             sev0, sev1, alv0, alv1, attv, mxv, s0a, s0b, s1a, s1b):
        cid = lax.axis_index("c")
        sid = lax.axis_index("s")
        wid = sid * _NC + cid
        pltpu.sync_copy(att_h, attv)

        def pair(k, mx):
            c0 = k * 2
            base0 = wid * _EPT + c0 * _K
            base1 = base0 + _K
            pltpu.sync_copy(src_h.at[pl.ds(base0, _K)], srcv0)
            pltpu.sync_copy(dst_h.at[pl.ds(base0, _K)], dstv0)
            cpa = pltpu.async_copy(xlr_h.at[srcv0], gsrc0, s0a)
            cpb = pltpu.async_copy(xlr_h.at[dstv0], gdst0, s0b)
            pltpu.sync_copy(se_h.at[pl.ds(base0, _K)], sev0)
            pltpu.sync_copy(src_h.at[pl.ds(base1, _K)], srcv1)
            pltpu.sync_copy(dst_h.at[pl.ds(base1, _K)], dstv1)
            cpc = pltpu.async_copy(xlr_h.at[srcv1], gsrc1, s1a)
            cpd = pltpu.async_copy(xlr_h.at[dstv1], gdst1, s1b)
            pltpu.sync_copy(se_h.at[pl.ds(base1, _K)], sev1)
            cpa.wait()
            cpb.wait()
            mx = _alpha_compute(gsrc0, gdst0, sev0, attv, alv0, mx)
            pltpu.sync_copy(alv0, alpha_h.at[pl.ds(base0, _K)])
            cpc.wait()
            cpd.wait()
            mx = _alpha_compute(gsrc1, gdst1, sev1, attv, alv1, mx)
            pltpu.sync_copy(alv1, alpha_h.at[pl.ds(base1, _K)])
            return mx

        mx = lax.fori_loop(0, _NCHUNK // 2, pair,
                           jnp.full((_L,), NEG, jnp.float32))

        base = wid * _EPT + (_NCHUNK - 1) * _K
        pltpu.sync_copy(src_h.at[pl.ds(base, _K)], srcv0)
        pltpu.sync_copy(dst_h.at[pl.ds(base, _K)], dstv0)
        cpa = pltpu.async_copy(xlr_h.at[srcv0], gsrc0, s0a)
        cpb = pltpu.async_copy(xlr_h.at[dstv0], gdst0, s0b)
        pltpu.sync_copy(se_h.at[pl.ds(base, _K)], sev0)
        cpa.wait()
        cpb.wait()
        mx = _alpha_compute(gsrc0, gdst0, sev0, attv, alv0, mx)
        pltpu.sync_copy(alv0, alpha_h.at[pl.ds(base, _K)])

        mxv[...] = mx
        pltpu.sync_copy(mxv, tmax_h.at[wid])

    return body(xlr, se, src, dst, att)


_AW = 2 * HID


def _scatter_compute(xlv, alv, gmax, scn, lane):
    for g in range(_K // _L):
        ex16 = jnp.exp(alv[pl.ds(g * _L, _L)] - gmax)
        for j in range(_L):
            i = g * _L + j
            exb = _shuf(ex16, jnp.full((_L,), j, jnp.int32))
            for f in range(HID // _L):
                scn[i, pl.ds(f * _L, _L)] = xlv[i, pl.ds(f * _L, _L)] * exb
            scn[i, pl.ds(HID, _L)] = jnp.where(lane == 0, exb, 0.0)


def _sc_scatter(xlr, src, dst, alpha, tmax, zn):
    @functools.partial(
        pl.kernel,
        out_type=jax.ShapeDtypeStruct((2 * N, _AW), jnp.float32),
        mesh=_MESH,
        scratch_types=[
            pltpu.VMEM((_K,), jnp.int32),
            pltpu.VMEM((_K,), jnp.int32),
            pltpu.VMEM((_K,), jnp.int32),
            pltpu.VMEM((_K,), jnp.int32),
            pltpu.VMEM((_K,), jnp.float32),
            pltpu.VMEM((_K,), jnp.float32),
            pltpu.VMEM((_K, 2 * HID), jnp.float32),
            pltpu.VMEM((_K, 2 * HID), jnp.float32),
            pltpu.VMEM((_K, _AW), jnp.float32),
            pltpu.VMEM((_K, _AW), jnp.float32),
            pltpu.VMEM((_NW, 16), jnp.float32),
            pltpu.VMEM_SHARED((N, _AW), jnp.float32),
            pltpu.SemaphoreType.DMA,
            pltpu.SemaphoreType.DMA,
            pltpu.SemaphoreType.DMA,
        ],
    )
    def body(xlr_h, src_h, dst_h, alpha_h, tmax_h, zn_h, acc_h,
             srcv0, srcv1, dstv0, dstv1, alv0, alv1, xlv0, xlv1,
             scn0, scn1, tmaxv, acc_s, sg0, sg1, sw0):
        cid = lax.axis_index("c")
        sid = lax.axis_index("s")
        wid = sid * _NC + cid

        pltpu.sync_copy(zn_h, scn0)
        for k in range(_CPY // _K):
            pltpu.sync_copy(scn0, acc_s.at[pl.ds(sid * _CSTEP + k * _K, _K)])

        zero = jnp.zeros((_L,), jnp.float32)
        for i in range(_K):
            for f in range(HID + _L, 2 * HID, _L):
                scn0[i, pl.ds(f, _L)] = zero
                scn1[i, pl.ds(f, _L)] = zero

        pltpu.sync_copy(tmax_h, tmaxv)
        mm = tmaxv[0, :]
        for r in range(1, _NW):
            mm = jnp.maximum(mm, tmaxv[r, :])
        gmax = _lane_max(mm)

        plsc.subcore_barrier()
        lane = _lane_iota()

        def pair(k, carry):
            c0 = k * 2
            base0 = wid * _EPT + c0 * _K
            base1 = base0 + _K
            pltpu.sync_copy(src_h.at[pl.ds(base0, _K)], srcv0)
            pltpu.sync_copy(dst_h.at[pl.ds(base0, _K)], dstv0)
            cpa = pltpu.async_copy(xlr_h.at[srcv0], xlv0, sg0)
            pltpu.sync_copy(alpha_h.at[pl.ds(base0, _K)], alv0)
            pltpu.sync_copy(src_h.at[pl.ds(base1, _K)], srcv1)
            pltpu.sync_copy(dst_h.at[pl.ds(base1, _K)], dstv1)
            cpb = pltpu.async_copy(xlr_h.at[srcv1], xlv1, sg1)
            pltpu.sync_copy(alpha_h.at[pl.ds(base1, _K)], alv1)
            cpa.wait()
            _scatter_compute(xlv0, alv0, gmax, scn0, lane)
            cps = pltpu.async_copy(scn0, acc_s.at[dstv0], sw0, add=True)
            cpb.wait()
            _scatter_compute(xlv1, alv1, gmax, scn1, lane)
            cps.wait()
            pltpu.sync_copy(scn1, acc_s.at[dstv1], add=True)
            return carry

        lax.fori_loop(0, _NCHUNK // 2, pair, jnp.int32(0))

        base = wid * _EPT + (_NCHUNK - 1) * _K
        pltpu.sync_copy(src_h.at[pl.ds(base, _K)], srcv0)
        pltpu.sync_copy(dst_h.at[pl.ds(base, _K)], dstv0)
        cpa = pltpu.async_copy(xlr_h.at[srcv0], xlv0, sg0)
        pltpu.sync_copy(alpha_h.at[pl.ds(base, _K)], alv0)
        cpa.wait()
        _scatter_compute(xlv0, alv0, gmax, scn0, lane)
        pltpu.sync_copy(scn0, acc_s.at[dstv0], add=True)

        plsc.subcore_barrier()

        for k in range(_CPY // _K):
            off = sid * _CSTEP + k * _K
            pltpu.sync_copy(acc_s.at[pl.ds(off, _K)], scn0)
            pltpu.sync_copy(scn0, acc_h.at[pl.ds(cid * N + off, _K)])

    return body(xlr, src, dst, alpha, tmax, zn)


def _edge_phase(xlr, se, src, dst, att, zn):
    alpha, tmax = _sc_alpha(xlr, se, src, dst, att)
    acc = _sc_scatter(xlr, src, dst, alpha, tmax, zn)
    return (acc[:N, :HID], acc[N:, :HID],
            acc[:N, HID:HID + 16], acc[N:, HID:HID + 16])


def kernel(x, edge_index, edge_attr, batch, params):
    src, dst = edge_index[0], edge_index[1]
    batchf = jnp.broadcast_to(batch.astype(jnp.float32)[:, None], (N, 8))
    zn = jnp.zeros((_K, _AW), jnp.float32)

    x = _linear(x, params["pre_node"]["W"], params["pre_node"]["b"], "silu",
                _NODE_BLK)
    ea = _linear(edge_attr, params["pre_edge"]["W"], params["pre_edge"]["b"],
                 "silu", _EDGE_BLK)

    for name in ("layer0", "layer1"):
        lp = params[name]
        cp = lp["conv"]
        se = _linear(ea, cp["lin_e_W"], jnp.zeros((HID,), jnp.float32), None,
                     _EDGE_BLK)
        W_lr = jnp.concatenate([cp["lin_l"]["W"], cp["lin_r"]["W"]], axis=0)
        b_lr = jnp.concatenate([cp["lin_l"]["b"], cp["lin_r"]["b"]])
        for _ in range(2):
            xlr = _linear(x, W_lr, b_lr, None, _NODE_BLK)
            n0, n1, d0, d1 = _edge_phase(xlr, se, src, dst, cp["att"], zn)
            x = _gru_node(n0, n1, d0, d1, cp["bias"], x, lp["gru"])
        x = _dgn_node(x, lp["norm"])

    out = _pool(x, batchf)
    gp = params["gconv"]
    xl_g = _linear(x, gp["lin_l"]["W"], gp["lin_l"]["b"], None, _NODE_BLK)
    for _ in range(2):
        xr_g = _linear(out, gp["lin_r"]["W"], gp["lin_r"]["b"], None, B)
        num, den = _ggat(xl_g, batchf, xr_g, gp["att"])
        out = _gru_graph(num, den, gp["bias"], out, params["ggru"])
    y = _dgn_graph(out, params["gnorm"])
    y = _linear(y, params["post0"]["W"], params["post0"]["b"], "silu", B)
    y = _linear(y, params["post1"]["W"], params["post1"]["b"], "silu", B)
    return _linear(y, params["out"]["W"], params["out"]["b"], None, B)


# speedup vs baseline: 8.1817x; 1.2319x over previous
import functools

import jax
import jax.numpy as jnp
from jax import lax
from jax.experimental import pallas as pl
from jax.experimental.pallas import tpu as pltpu
from jax.experimental.pallas import tpu_sc as plsc

N = 10000
E = 320000
HID = 64
B = 64
GROUPS = 10
LAMDA = 0.01
EPS = 1e-5
NEG = -1e30

_NODE_BLK = 400
_EDGE_BLK = 2000


def _silu(v):
    return v * jax.nn.sigmoid(v)


def _elu(v):
    return jnp.where(v > 0, v, jnp.exp(jnp.minimum(v, 0.0)) - 1.0)


def _pad8(v):
    return jnp.broadcast_to(v[None, :], (8, v.shape[0]))


def _lin_body(act, x_ref, wt_ref, b_ref, o_ref):
    v = jnp.dot(x_ref[...], wt_ref[...], preferred_element_type=jnp.float32)
    v = v + b_ref[0:1, :]
    if act == "silu":
        v = _silu(v)
    o_ref[...] = v


def _linear(x, W, b, act, blk):
    rows, din = x.shape
    h = W.shape[0]
    grid = rows // blk
    return pl.pallas_call(
        functools.partial(_lin_body, act),
        grid=(grid,),
        in_specs=[
            pl.BlockSpec((blk, din), lambda i: (i, 0)),
            pl.BlockSpec((din, h), lambda i: (0, 0)),
            pl.BlockSpec((8, h), lambda i: (0, 0)),
        ],
        out_specs=pl.BlockSpec((blk, h), lambda i: (i, 0)),
        out_shape=jax.ShapeDtypeStruct((rows, h), jnp.float32),
    )(x, W.T, _pad8(b))


def _gru_math(g, hprev, wr, wz, wn, ur, uz, un, br, bz, bn, cr, cz, cn):
    ir = jnp.dot(g, wr, preferred_element_type=jnp.float32) + br[0:1, :]
    iz = jnp.dot(g, wz, preferred_element_type=jnp.float32) + bz[0:1, :]
    inn = jnp.dot(g, wn, preferred_element_type=jnp.float32) + bn[0:1, :]
    hr = jnp.dot(hprev, ur, preferred_element_type=jnp.float32) + cr[0:1, :]
    hz = jnp.dot(hprev, uz, preferred_element_type=jnp.float32) + cz[0:1, :]
    hn = jnp.dot(hprev, un, preferred_element_type=jnp.float32) + cn[0:1, :]
    r = jax.nn.sigmoid(ir + hr)
    z = jax.nn.sigmoid(iz + hz)
    n = jnp.tanh(inn + r * hn)
    return jnp.maximum((1.0 - z) * n + z * hprev, 0.0)


def _gru_node_body(num0_ref, num1_ref, den0_ref, den1_ref, bias_ref, x_ref,
                   wr, wz, wn, ur, uz, un, br, bz, bn, cr, cz, cn, o_ref):
    num = num0_ref[...] + num1_ref[...]
    den = den0_ref[...][:, 0:1] + den1_ref[...][:, 0:1]
    g = _elu(num / (den + 1e-16) + bias_ref[0:1, :])
    o_ref[...] = _gru_math(g, x_ref[...], wr[...], wz[...], wn[...],
                           ur[...], uz[...], un[...], br[...], bz[...],
                           bn[...], cr[...], cz[...], cn[...])


def _split_gru(p):
    Wih, Whh = p["W_ih"], p["W_hh"]
    bih, bhh = p["b_ih"], p["b_hh"]
    outs = []
    for i in range(3):
        outs.append(Wih[i * HID:(i + 1) * HID].T)
    for i in range(3):
        outs.append(Whh[i * HID:(i + 1) * HID].T)
    for i in range(3):
        outs.append(_pad8(bih[i * HID:(i + 1) * HID]))
    for i in range(3):
        outs.append(_pad8(bhh[i * HID:(i + 1) * HID]))
    return outs


def _gru_node(num0, num1, den0, den1, bias, x, gp):
    blk = _NODE_BLK
    grid = N // blk
    wmats = _split_gru(gp)
    full = lambda s: pl.BlockSpec(s, lambda i: (0, 0))
    rowspec = lambda w: pl.BlockSpec((blk, w), lambda i: (i, 0))
    return pl.pallas_call(
        _gru_node_body,
        grid=(grid,),
        in_specs=[rowspec(HID), rowspec(HID), rowspec(16), rowspec(16),
                  full((8, HID)), rowspec(HID)]
                 + [full((HID, HID))] * 6 + [full((8, HID))] * 6,
        out_specs=rowspec(HID),
        out_shape=jax.ShapeDtypeStruct((N, HID), jnp.float32),
    )(num0, num1, den0, den1, _pad8(bias), x, *wmats)


def _gru_graph_body(num_ref, den_ref, bias_ref, h_ref,
                    wr, wz, wn, ur, uz, un, br, bz, bn, cr, cz, cn, o_ref):
    g = _elu(num_ref[...] / (den_ref[...][:, 0:1] + 1e-16) + bias_ref[0:1, :])
    o_ref[...] = _gru_math(g, h_ref[...], wr[...], wz[...], wn[...],
                           ur[...], uz[...], un[...], br[...], bz[...],
                           bn[...], cr[...], cz[...], cn[...])


def _gru_graph(num, den, bias, h, gp):
    wmats = _split_gru(gp)
    return pl.pallas_call(
        _gru_graph_body,
        out_shape=jax.ShapeDtypeStruct((B, HID), jnp.float32),
    )(num, den, _pad8(bias), h, *wmats)


def _softmax_s(xb, wnt, bn16):
    logits = jnp.dot(xb, wnt, preferred_element_type=jnp.float32) + bn16[0:1, :]
    m = jnp.max(logits, axis=1, keepdims=True)
    s = jnp.exp(logits - m)
    return s / jnp.sum(s, axis=1, keepdims=True)


def _dgn_stats_body(x_ref, wnt_ref, bn_ref, m_ref, q_ref):
    i = pl.program_id(0)
    xb = x_ref[...]
    s = _softmax_s(xb, wnt_ref[...], bn_ref[...])
    mp = lax.dot_general(s, xb, (((0,), (0,)), ((), ())),
                         preferred_element_type=jnp.float32)
    qp = lax.dot_general(s * s, xb * xb, (((0,), (0,)), ((), ())),
                         preferred_element_type=jnp.float32)

    @pl.when(i == 0)
    def _():
        m_ref[...] = jnp.zeros_like(m_ref)
        q_ref[...] = jnp.zeros_like(q_ref)

    m_ref[...] += mp
    q_ref[...] += qp


def _dgn_apply_body(nrows, x_ref, wnt_ref, bn_ref, m_ref, q_ref,
                    bw_ref, bb_ref, o_ref):
    xb = x_ref[...]
    s = _softmax_s(xb, wnt_ref[...], bn_ref[...])
    mean = m_ref[...] * (1.0 / nrows)
    var = q_ref[...] * (1.0 / nrows) - mean * mean
    rstd = lax.rsqrt(var + EPS)
    wsc = bw_ref[...] * rstd
    a = jnp.dot(s, wsc, preferred_element_type=jnp.float32)
    c = jnp.sum(mean * wsc - bb_ref[...], axis=0, keepdims=True)
    o_ref[...] = xb + LAMDA * (xb * a - c)


def _dgn_prep(p):
    W, b = p["lin"]["W"], p["lin"]["b"]
    wnt = jnp.zeros((HID, 16), jnp.float32).at[:, :GROUPS].set(W.T)
    bn16 = jnp.full((16,), NEG, jnp.float32).at[:GROUPS].set(b)
    bw = jnp.zeros((16, HID), jnp.float32).at[:GROUPS].set(
        p["bn_w"].reshape(GROUPS, HID))
    bb = jnp.zeros((16, HID), jnp.float32).at[:GROUPS].set(
        p["bn_b"].reshape(GROUPS, HID))
    return wnt, _pad8(bn16), bw, bb


def _dgn_node(x, p):
    wnt, bn16, bw, bb = _dgn_prep(p)
    blk = _NODE_BLK
    grid = N // blk
    full = lambda s: pl.BlockSpec(s, lambda i: (0, 0))
    m, q = pl.pallas_call(
        _dgn_stats_body,
        grid=(grid,),
        in_specs=[pl.BlockSpec((blk, HID), lambda i: (i, 0)),
                  full((HID, 16)), full((8, 16))],
        out_specs=[full((16, HID)), full((16, HID))],
        out_shape=[jax.ShapeDtypeStruct((16, HID), jnp.float32)] * 2,
    )(x, wnt, bn16)
    return pl.pallas_call(
        functools.partial(_dgn_apply_body, float(N)),
        grid=(grid,),
        in_specs=[pl.BlockSpec((blk, HID), lambda i: (i, 0)),
                  full((HID, 16)), full((8, 16)), full((16, HID)),
                  full((16, HID)), full((16, HID)), full((16, HID))],
        out_specs=pl.BlockSpec((blk, HID), lambda i: (i, 0)),
        out_shape=jax.ShapeDtypeStruct((N, HID), jnp.float32),
    )(x, wnt, bn16, m, q, bw, bb)


def _dgn_graph_body(x_ref, wnt_ref, bn_ref, bw_ref, bb_ref, o_ref):
    xb = x_ref[...]
    s = _softmax_s(xb, wnt_ref[...], bn_ref[...])
    mp = lax.dot_general(s, xb, (((0,), (0,)), ((), ())),
                         preferred_element_type=jnp.float32)
    qp = lax.dot_general(s * s, xb * xb, (((0,), (0,)), ((), ())),
                         preferred_element_type=jnp.float32)
    mean = mp * (1.0 / B)
    var = qp * (1.0 / B) - mean * mean
    rstd = lax.rsqrt(var + EPS)
    wsc = bw_ref[...] * rstd
    a = jnp.dot(s, wsc, preferred_element_type=jnp.float32)
    c = jnp.sum(mean * wsc - bb_ref[...], axis=0, keepdims=True)
    o_ref[...] = xb + LAMDA * (xb * a - c)


def _dgn_graph(x, p):
    wnt, bn16, bw, bb = _dgn_prep(p)
    return pl.pallas_call(
        _dgn_graph_body,
        out_shape=jax.ShapeDtypeStruct((B, HID), jnp.float32),
    )(x, wnt, bn16, bw, bb)


def _pool_body(nblocks, bf_ref, x_ref, o_ref):
    i = pl.program_id(0)
    iota = lax.broadcasted_iota(jnp.int32, (1, B), 1).astype(jnp.float32)
    onehot = (bf_ref[...][:, 0:1] == iota).astype(jnp.float32)
    part = lax.dot_general(onehot, x_ref[...], (((0,), (0,)), ((), ())),
                           preferred_element_type=jnp.float32)

    @pl.when(i == 0)
    def _():
        o_ref[...] = jnp.zeros_like(o_ref)

    o_ref[...] += part

    @pl.when(i == nblocks - 1)
    def _():
        o_ref[...] = jnp.maximum(o_ref[...], 0.0)


def _pool(x, batchf):
    blk = _NODE_BLK
    grid = N // blk
    return pl.pallas_call(
        functools.partial(_pool_body, grid),
        grid=(grid,),
        in_specs=[pl.BlockSpec((blk, 8), lambda i: (i, 0)),
                  pl.BlockSpec((blk, HID), lambda i: (i, 0))],
        out_specs=pl.BlockSpec((B, HID), lambda i: (0, 0)),
        out_shape=jax.ShapeDtypeStruct((B, HID), jnp.float32),
    )(batchf, x)


def _ggat_alpha(xlb, bf, xr, att):
    iota = lax.broadcasted_iota(jnp.int32, (1, B), 1).astype(jnp.float32)
    onehot = (bf[:, 0:1] == iota).astype(jnp.float32)
    e = xlb + jnp.dot(onehot, xr, preferred_element_type=jnp.float32)
    e = jnp.where(e > 0, e, 0.01 * e)
    alpha = jnp.dot(e, att, preferred_element_type=jnp.float32)
    return onehot, alpha


def _ggat1_body(nblocks, xl_ref, bf_ref, xr_ref, att_ref, amax_ref):
    i = pl.program_id(0)
    onehot, alpha = _ggat_alpha(xl_ref[...], bf_ref[...], xr_ref[...],
                                att_ref[...][:, 0:1])
    masked = jnp.where(onehot > 0, alpha, NEG)
    pmax = jnp.max(masked, axis=0, keepdims=True)

    @pl.when(i == 0)
    def _():
        amax_ref[...] = jnp.full_like(amax_ref, NEG)

    amax_ref[...] = jnp.maximum(amax_ref[...], jnp.broadcast_to(pmax, (8, B)))


def _ggat2_body(xl_ref, bf_ref, xr_ref, att_ref, amax_ref, num_ref, den_ref):
    i = pl.program_id(0)
    onehot, alpha = _ggat_alpha(xl_ref[...], bf_ref[...], xr_ref[...],
                                att_ref[...][:, 0:1])
    am = amax_ref[...][0:1, :]
    am = jnp.where(am < -1e29, 0.0, am)
    amrow = jnp.sum(onehot * am, axis=1, keepdims=True)
    ex = jnp.exp(alpha - amrow)
    np_ = lax.dot_general(onehot, ex * xl_ref[...], (((0,), (0,)), ((), ())),
                          preferred_element_type=jnp.float32)
    dp = lax.dot_general(onehot, jnp.broadcast_to(ex, ex.shape[:1] + (8,)),
                         (((0,), (0,)), ((), ())),
                         preferred_element_type=jnp.float32)

    @pl.when(i == 0)
    def _():
        num_ref[...] = jnp.zeros_like(num_ref)
        den_ref[...] = jnp.zeros_like(den_ref)

    num_ref[...] += np_
    den_ref[...] += dp


def _ggat(xl, batchf, xr, att):
    blk = _NODE_BLK
    grid = N // blk
    full = lambda s: pl.BlockSpec(s, lambda i: (0, 0))
    att2 = jnp.broadcast_to(att[:, None], (HID, 8))
    amax = pl.pallas_call(
        functools.partial(_ggat1_body, grid),
        grid=(grid,),
        in_specs=[pl.BlockSpec((blk, HID), lambda i: (i, 0)),
                  pl.BlockSpec((blk, 8), lambda i: (i, 0)),
                  full((B, HID)), full((HID, 8))],
        out_specs=full((8, B)),
        out_shape=jax.ShapeDtypeStruct((8, B), jnp.float32),
    )(xl, batchf, xr, att2)
    num, den = pl.pallas_call(
        _ggat2_body,
        grid=(grid,),
        in_specs=[pl.BlockSpec((blk, HID), lambda i: (i, 0)),
                  pl.BlockSpec((blk, 8), lambda i: (i, 0)),
                  full((B, HID)), full((HID, 8)), full((8, B))],
        out_specs=[full((B, HID)), full((B, 8))],
        out_shape=[jax.ShapeDtypeStruct((B, HID), jnp.float32),
                   jax.ShapeDtypeStruct((B, 8), jnp.float32)],
    )(xl, batchf, xr, att2, amax)
    return num, den


_NC, _NS, _L = 2, 16, 16
_NW = _NC * _NS
_EPT = E // _NW
_K = 80
_NCHUNK = _EPT // _K
_CPY = 640
_CSTEP = 624

_MESH = plsc.VectorSubcoreMesh(core_axis_name="c", subcore_axis_name="s")


def _lane_iota():
    return lax.broadcasted_iota(jnp.int32, (_L,), 0)


def _shuf(v, idx):
    dnums = lax.GatherDimensionNumbers(
        offset_dims=(), collapsed_slice_dims=(0,), start_index_map=(0,))
    return lax.gather(v, idx[:, None], dnums, slice_sizes=(1,),
                      mode=lax.GatherScatterMode.PROMISE_IN_BOUNDS)


def _lane_sum(v):
    lane = _lane_iota()
    for s in (8, 4, 2, 1):
        v = v + _shuf(v, lane ^ s)
    return v


def _lane_max(v):
    lane = _lane_iota()
    for s in (8, 4, 2, 1):
        v = jnp.maximum(v, _shuf(v, lane ^ s))
    return v


_SBK = 4
_SBE = _SBK * _K
_NSB = (_NCHUNK - 1) // _SBK


def _alpha_compute(gsrc, gdst, sev, attv, alv, off, mx):
    for g in range(_K // _L):
        av = jnp.zeros((_L,), jnp.float32)
        for j in range(_L):
            i = g * _L + j
            acc = jnp.zeros((_L,), jnp.float32)
            for f in range(HID // _L):
                v = (gsrc[i, pl.ds(f * _L, _L)]
                     + gdst[i, pl.ds(HID + f * _L, _L)]
                     + sev[i, pl.ds(f * _L, _L)])
                v = jnp.maximum(v, 0.0) + 0.01 * jnp.minimum(v, 0.0)
                acc = acc + v * attv[pl.ds(f * _L, _L)]
            a = _lane_sum(acc)
            av = jnp.where(_lane_iota() == j, a, av)
        alv[pl.ds(off + g * _L, _L)] = av
        mx = jnp.maximum(mx, av)
    return mx


def _sc_alpha(xlr, se, src, dst, att):
    @functools.partial(
        pl.kernel,
        out_type=[jax.ShapeDtypeStruct((E,), jnp.float32),
                  jax.ShapeDtypeStruct((_NW, 16), jnp.float32)],
        mesh=_MESH,
        scratch_types=[
            pltpu.VMEM((_SBE,), jnp.int32),
            pltpu.VMEM((_SBE,), jnp.int32),
            pltpu.VMEM((_SBE,), jnp.float32),
            pltpu.VMEM((_K, 2 * HID), jnp.float32),
            pltpu.VMEM((_K, 2 * HID), jnp.float32),
            pltpu.VMEM((_K, 2 * HID), jnp.float32),
            pltpu.VMEM((_K, 2 * HID), jnp.float32),
            pltpu.VMEM((_K, HID), jnp.float32),
            pltpu.VMEM((_K, HID), jnp.float32),
            pltpu.VMEM((HID,), jnp.float32),
            pltpu.VMEM((16,), jnp.float32),
            pltpu.SemaphoreType.DMA,
            pltpu.SemaphoreType.DMA,
            pltpu.SemaphoreType.DMA,
            pltpu.SemaphoreType.DMA,
            pltpu.SemaphoreType.DMA,
            pltpu.SemaphoreType.DMA,
            pltpu.SemaphoreType.DMA,
        ],
    )
    def body(xlr_h, se_h, src_h, dst_h, att_h, alpha_h, tmax_h,
             srcb, dstb, albuf, gsrc0, gsrc1, gdst0, gdst1,
             sev0, sev1, attv, mxv, six, s0a, s0b, s1a, s1b, sse0, sse1):
        cid = lax.axis_index("c")
        sid = lax.axis_index("s")
        wid = sid * _NC + cid
        pltpu.sync_copy(att_h, attv)
        gsrc = (gsrc0, gsrc1)
        gdst = (gdst0, gdst1)
        sev = (sev0, sev1)
        sg = ((s0a, s0b), (s1a, s1b))
        sse = (sse0, sse1)

        def sblock(sb, mx):
            base = wid * _EPT + sb * _SBE
            ix1 = pltpu.async_copy(src_h.at[pl.ds(base, _SBE)], srcb, six)
            ix2 = pltpu.async_copy(dst_h.at[pl.ds(base, _SBE)], dstb, six)
            secp = {}
            for j in range(2):
                secp[j] = pltpu.async_copy(
                    se_h.at[pl.ds(base + j * _K, _K)], sev[j], sse[j])
            ix1.wait()
            ix2.wait()
            gcp = {}
            for j in range(2):
                b = j % 2
                gcp[j] = (
                    pltpu.async_copy(
                        xlr_h.at[srcb.at[pl.ds(j * _K, _K)]], gsrc[b],
                        sg[b][0]),
                    pltpu.async_copy(
                        xlr_h.at[dstb.at[pl.ds(j * _K, _K)]], gdst[b],
                        sg[b][1]))
            for j in range(_SBK):
                b = j % 2
                gcp[j][0].wait()
                gcp[j][1].wait()
                secp[j].wait()
                mx = _alpha_compute(gsrc[b], gdst[b], sev[b], attv,
                                    albuf, j * _K, mx)
                if j + 2 < _SBK:
                    gcp[j + 2] = (
                        pltpu.async_copy(
                            xlr_h.at[srcb.at[pl.ds((j + 2) * _K, _K)]],
                            gsrc[b], sg[b][0]),
                        pltpu.async_copy(
                            xlr_h.at[dstb.at[pl.ds((j + 2) * _K, _K)]],
                            gdst[b], sg[b][1]))
                    secp[j + 2] = pltpu.async_copy(
                        se_h.at[pl.ds(base + (j + 2) * _K, _K)], sev[b],
                        sse[b])
            pltpu.sync_copy(albuf, alpha_h.at[pl.ds(base, _SBE)])
            return mx

        mx = lax.fori_loop(0, _NSB, sblock,
                           jnp.full((_L,), NEG, jnp.float32))

        base = wid * _EPT + _NSB * _SBE
        pltpu.sync_copy(src_h.at[pl.ds(base, _K)], srcb.at[pl.ds(0, _K)])
        pltpu.sync_copy(dst_h.at[pl.ds(base, _K)], dstb.at[pl.ds(0, _K)])
        cpa = pltpu.async_copy(xlr_h.at[srcb.at[pl.ds(0, _K)]], gsrc0, s0a)
        cpb = pltpu.async_copy(xlr_h.at[dstb.at[pl.ds(0, _K)]], gdst0, s0b)
        pltpu.sync_copy(se_h.at[pl.ds(base, _K)], sev0)
        cpa.wait()
        cpb.wait()
        mx = _alpha_compute(gsrc0, gdst0, sev0, attv, albuf, 0, mx)
        pltpu.sync_copy(albuf.at[pl.ds(0, _K)], alpha_h.at[pl.ds(base, _K)])

        mxv[...] = mx
        pltpu.sync_copy(mxv, tmax_h.at[wid])

    return body(xlr, se, src, dst, att)


_AW = 2 * HID


def _scatter_compute(xlv, alv, off, gmax, scn, lane):
    for g in range(_K // _L):
        ex16 = jnp.exp(alv[pl.ds(off + g * _L, _L)] - gmax)
        for j in range(_L):
            i = g * _L + j
            exb = _shuf(ex16, jnp.full((_L,), j, jnp.int32))
            for f in range(HID // _L):
                scn[i, pl.ds(f * _L, _L)] = xlv[i, pl.ds(f * _L, _L)] * exb
            scn[i, pl.ds(HID, _L)] = jnp.where(lane == 0, exb, 0.0)


def _sc_scatter(xlr, src, dst, alpha, tmax, zn):
    @functools.partial(
        pl.kernel,
        out_type=jax.ShapeDtypeStruct((2 * N, _AW), jnp.float32),
        mesh=_MESH,
        scratch_types=[
            pltpu.VMEM((_SBE,), jnp.int32),
            pltpu.VMEM((_SBE,), jnp.float32),
            pltpu.VMEM((_K,), jnp.int32),
            pltpu.VMEM((_K,), jnp.int32),
            pltpu.VMEM((_K,), jnp.int32),
            pltpu.VMEM((_K,), jnp.int32),
            pltpu.VMEM((_K, 2 * HID), jnp.float32),
            pltpu.VMEM((_K, 2 * HID), jnp.float32),
            pltpu.VMEM((_K, _AW), jnp.float32),
            pltpu.VMEM((_K, _AW), jnp.float32),
            pltpu.VMEM((_NW, 16), jnp.float32),
            pltpu.VMEM_SHARED((N, _AW), jnp.float32),
            pltpu.SemaphoreType.DMA,
            pltpu.SemaphoreType.DMA,
            pltpu.SemaphoreType.DMA,
            pltpu.SemaphoreType.DMA,
            pltpu.SemaphoreType.DMA,
        ],
    )
    def body(xlr_h, src_h, dst_h, alpha_h, tmax_h, zn_h, acc_h,
             srcb, albuf, dstv0, dstv1, dstv2, dstv3, xlv0, xlv1,
             scn0, scn1, tmaxv, acc_s, six, sg0, sg1, sw0, sw1):
        cid = lax.axis_index("c")
        sid = lax.axis_index("s")
        wid = sid * _NC + cid

        pltpu.sync_copy(zn_h, scn0)
        for k in range(_CPY // _K):
            pltpu.sync_copy(scn0, acc_s.at[pl.ds(sid * _CSTEP + k * _K, _K)])

        zero = jnp.zeros((_L,), jnp.float32)
        for i in range(_K):
            for f in range(HID + _L, 2 * HID, _L):
                scn0[i, pl.ds(f, _L)] = zero
                scn1[i, pl.ds(f, _L)] = zero

        pltpu.sync_copy(tmax_h, tmaxv)
        mm = tmaxv[0, :]
        for r in range(1, _NW):
            mm = jnp.maximum(mm, tmaxv[r, :])
        gmax = _lane_max(mm)

        plsc.subcore_barrier()
        lane = _lane_iota()
        dstv = (dstv0, dstv1, dstv2, dstv3)
        xlv = (xlv0, xlv1)
        scn = (scn0, scn1)
        sg = (sg0, sg1)
        sw = (sw0, sw1)

        def sblock(sb, carry):
            base = wid * _EPT + sb * _SBE
            ixcp = [pltpu.async_copy(src_h.at[pl.ds(base, _SBE)], srcb, six),
                    pltpu.async_copy(alpha_h.at[pl.ds(base, _SBE)], albuf,
                                     six)]
            for j in range(_SBK):
                ixcp.append(pltpu.async_copy(
                    dst_h.at[pl.ds(base + j * _K, _K)], dstv[j], six))
            for cp in ixcp:
                cp.wait()
            gcp = {}
            for j in range(2):
                gcp[j] = pltpu.async_copy(
                    xlr_h.at[srcb.at[pl.ds(j * _K, _K)]], xlv[j], sg[j])
            scp = {}
            for j in range(_SBK):
                b = j % 2
                gcp[j].wait()
                if j >= 2:
                    scp[j - 2].wait()
                _scatter_compute(xlv[b], albuf, j * _K, gmax, scn[b], lane)
                scp[j] = pltpu.async_copy(scn[b], acc_s.at[dstv[j]], sw[b],
                                          add=True)
                if j + 2 < _SBK:
                    gcp[j + 2] = pltpu.async_copy(
                        xlr_h.at[srcb.at[pl.ds((j + 2) * _K, _K)]], xlv[b],
                        sg[b])
            scp[_SBK - 2].wait()
            scp[_SBK - 1].wait()
            return carry

        lax.fori_loop(0, _NSB, sblock, jnp.int32(0))

        base = wid * _EPT + _NSB * _SBE
        pltpu.sync_copy(src_h.at[pl.ds(base, _K)], srcb.at[pl.ds(0, _K)])
        pltpu.sync_copy(dst_h.at[pl.ds(base, _K)], dstv0)
        cpa = pltpu.async_copy(xlr_h.at[srcb.at[pl.ds(0, _K)]], xlv0, sg0)
        pltpu.sync_copy(alpha_h.at[pl.ds(base, _K)], albuf.at[pl.ds(0, _K)])
        cpa.wait()
        _scatter_compute(xlv0, albuf, 0, gmax, scn0, lane)
        pltpu.sync_copy(scn0, acc_s.at[dstv0], add=True)

        plsc.subcore_barrier()

        for k in range(_CPY // _K):
            off = sid * _CSTEP + k * _K
            pltpu.sync_copy(acc_s.at[pl.ds(off, _K)], scn0)
            pltpu.sync_copy(scn0, acc_h.at[pl.ds(cid * N + off, _K)])

    return body(xlr, src, dst, alpha, tmax, zn)


def _edge_phase(xlr, se, src, dst, att, zn):
    alpha, tmax = _sc_alpha(xlr, se, src, dst, att)
    acc = _sc_scatter(xlr, src, dst, alpha, tmax, zn)
    return (acc[:N, :HID], acc[N:, :HID],
            acc[:N, HID:HID + 16], acc[N:, HID:HID + 16])


def kernel(x, edge_index, edge_attr, batch, params):
    src, dst = edge_index[0], edge_index[1]
    batchf = jnp.broadcast_to(batch.astype(jnp.float32)[:, None], (N, 8))
    zn = jnp.zeros((_K, _AW), jnp.float32)

    x = _linear(x, params["pre_node"]["W"], params["pre_node"]["b"], "silu",
                _NODE_BLK)
    ea = _linear(edge_attr, params["pre_edge"]["W"], params["pre_edge"]["b"],
                 "silu", _EDGE_BLK)

    for name in ("layer0", "layer1"):
        lp = params[name]
        cp = lp["conv"]
        se = _linear(ea, cp["lin_e_W"], jnp.zeros((HID,), jnp.float32), None,
                     _EDGE_BLK)
        W_lr = jnp.concatenate([cp["lin_l"]["W"], cp["lin_r"]["W"]], axis=0)
        b_lr = jnp.concatenate([cp["lin_l"]["b"], cp["lin_r"]["b"]])
        for _ in range(2):
            xlr = _linear(x, W_lr, b_lr, None, _NODE_BLK)
            n0, n1, d0, d1 = _edge_phase(xlr, se, src, dst, cp["att"], zn)
            x = _gru_node(n0, n1, d0, d1, cp["bias"], x, lp["gru"])
        x = _dgn_node(x, lp["norm"])

    out = _pool(x, batchf)
    gp = params["gconv"]
    xl_g = _linear(x, gp["lin_l"]["W"], gp["lin_l"]["b"], None, _NODE_BLK)
    for _ in range(2):
        xr_g = _linear(out, gp["lin_r"]["W"], gp["lin_r"]["b"], None, B)
        num, den = _ggat(xl_g, batchf, xr_g, gp["att"])
        out = _gru_graph(num, den, gp["bias"], out, params["ggru"])
    y = _dgn_graph(out, params["gnorm"])
    y = _linear(y, params["post0"]["W"], params["post0"]["b"], "silu", B)
    y = _linear(y, params["post1"]["W"], params["post1"]["b"], "silu", B)
    return _linear(y, params["out"]["W"], params["out"]["b"], None, B)


# speedup vs baseline: 8.3221x; 1.0172x over previous
import functools

import jax
import jax.numpy as jnp
from jax import lax
from jax.experimental import pallas as pl
from jax.experimental.pallas import tpu as pltpu
from jax.experimental.pallas import tpu_sc as plsc

N = 10000
E = 320000
HID = 64
B = 64
GROUPS = 10
LAMDA = 0.01
EPS = 1e-5
NEG = -1e30

_NODE_BLK = 400
_EDGE_BLK = 2000


def _silu(v):
    return v * jax.nn.sigmoid(v)


def _elu(v):
    return jnp.where(v > 0, v, jnp.exp(jnp.minimum(v, 0.0)) - 1.0)


def _pad8(v):
    return jnp.broadcast_to(v[None, :], (8, v.shape[0]))


def _lin_body(act, x_ref, wt_ref, b_ref, o_ref):
    v = jnp.dot(x_ref[...], wt_ref[...], preferred_element_type=jnp.float32)
    v = v + b_ref[0:1, :]
    if act == "silu":
        v = _silu(v)
    o_ref[...] = v


def _linear(x, W, b, act, blk):
    rows, din = x.shape
    h = W.shape[0]
    grid = rows // blk
    return pl.pallas_call(
        functools.partial(_lin_body, act),
        grid=(grid,),
        in_specs=[
            pl.BlockSpec((blk, din), lambda i: (i, 0)),
            pl.BlockSpec((din, h), lambda i: (0, 0)),
            pl.BlockSpec((8, h), lambda i: (0, 0)),
        ],
        out_specs=pl.BlockSpec((blk, h), lambda i: (i, 0)),
        out_shape=jax.ShapeDtypeStruct((rows, h), jnp.float32),
    )(x, W.T, _pad8(b))


def _gru_math(g, hprev, wr, wz, wn, ur, uz, un, br, bz, bn, cr, cz, cn):
    ir = jnp.dot(g, wr, preferred_element_type=jnp.float32) + br[0:1, :]
    iz = jnp.dot(g, wz, preferred_element_type=jnp.float32) + bz[0:1, :]
    inn = jnp.dot(g, wn, preferred_element_type=jnp.float32) + bn[0:1, :]
    hr = jnp.dot(hprev, ur, preferred_element_type=jnp.float32) + cr[0:1, :]
    hz = jnp.dot(hprev, uz, preferred_element_type=jnp.float32) + cz[0:1, :]
    hn = jnp.dot(hprev, un, preferred_element_type=jnp.float32) + cn[0:1, :]
    r = jax.nn.sigmoid(ir + hr)
    z = jax.nn.sigmoid(iz + hz)
    n = jnp.tanh(inn + r * hn)
    return jnp.maximum((1.0 - z) * n + z * hprev, 0.0)


def _gru_node_body(num0_ref, num1_ref, den0_ref, den1_ref, bias_ref, x_ref,
                   wr, wz, wn, ur, uz, un, br, bz, bn, cr, cz, cn, o_ref):
    num = num0_ref[...] + num1_ref[...]
    den = den0_ref[...][:, 0:1] + den1_ref[...][:, 0:1]
    g = _elu(num / (den + 1e-16) + bias_ref[0:1, :])
    o_ref[...] = _gru_math(g, x_ref[...], wr[...], wz[...], wn[...],
                           ur[...], uz[...], un[...], br[...], bz[...],
                           bn[...], cr[...], cz[...], cn[...])


def _split_gru(p):
    Wih, Whh = p["W_ih"], p["W_hh"]
    bih, bhh = p["b_ih"], p["b_hh"]
    outs = []
    for i in range(3):
        outs.append(Wih[i * HID:(i + 1) * HID].T)
    for i in range(3):
        outs.append(Whh[i * HID:(i + 1) * HID].T)
    for i in range(3):
        outs.append(_pad8(bih[i * HID:(i + 1) * HID]))
    for i in range(3):
        outs.append(_pad8(bhh[i * HID:(i + 1) * HID]))
    return outs


def _gru_node(num0, num1, den0, den1, bias, x, gp):
    blk = _NODE_BLK
    grid = N // blk
    wmats = _split_gru(gp)
    full = lambda s: pl.BlockSpec(s, lambda i: (0, 0))
    rowspec = lambda w: pl.BlockSpec((blk, w), lambda i: (i, 0))
    return pl.pallas_call(
        _gru_node_body,
        grid=(grid,),
        in_specs=[rowspec(HID), rowspec(HID), rowspec(16), rowspec(16),
                  full((8, HID)), rowspec(HID)]
                 + [full((HID, HID))] * 6 + [full((8, HID))] * 6,
        out_specs=rowspec(HID),
        out_shape=jax.ShapeDtypeStruct((N, HID), jnp.float32),
    )(num0, num1, den0, den1, _pad8(bias), x, *wmats)


def _gru_graph_body(num_ref, den_ref, bias_ref, h_ref,
                    wr, wz, wn, ur, uz, un, br, bz, bn, cr, cz, cn, o_ref):
    g = _elu(num_ref[...] / (den_ref[...][:, 0:1] + 1e-16) + bias_ref[0:1, :])
    o_ref[...] = _gru_math(g, h_ref[...], wr[...], wz[...], wn[...],
                           ur[...], uz[...], un[...], br[...], bz[...],
                           bn[...], cr[...], cz[...], cn[...])


def _gru_graph(num, den, bias, h, gp):
    wmats = _split_gru(gp)
    return pl.pallas_call(
        _gru_graph_body,
        out_shape=jax.ShapeDtypeStruct((B, HID), jnp.float32),
    )(num, den, _pad8(bias), h, *wmats)


def _softmax_s(xb, wnt, bn16):
    logits = jnp.dot(xb, wnt, preferred_element_type=jnp.float32) + bn16[0:1, :]
    m = jnp.max(logits, axis=1, keepdims=True)
    s = jnp.exp(logits - m)
    return s / jnp.sum(s, axis=1, keepdims=True)


def _dgn_stats_body(x_ref, wnt_ref, bn_ref, m_ref, q_ref):
    i = pl.program_id(0)
    xb = x_ref[...]
    s = _softmax_s(xb, wnt_ref[...], bn_ref[...])
    mp = lax.dot_general(s, xb, (((0,), (0,)), ((), ())),
                         preferred_element_type=jnp.float32)
    qp = lax.dot_general(s * s, xb * xb, (((0,), (0,)), ((), ())),
                         preferred_element_type=jnp.float32)

    @pl.when(i == 0)
    def _():
        m_ref[...] = jnp.zeros_like(m_ref)
        q_ref[...] = jnp.zeros_like(q_ref)

    m_ref[...] += mp
    q_ref[...] += qp


def _dgn_apply_body(nrows, x_ref, wnt_ref, bn_ref, m_ref, q_ref,
                    bw_ref, bb_ref, o_ref):
    xb = x_ref[...]
    s = _softmax_s(xb, wnt_ref[...], bn_ref[...])
    mean = m_ref[...] * (1.0 / nrows)
    var = q_ref[...] * (1.0 / nrows) - mean * mean
    rstd = lax.rsqrt(var + EPS)
    wsc = bw_ref[...] * rstd
    a = jnp.dot(s, wsc, preferred_element_type=jnp.float32)
    c = jnp.sum(mean * wsc - bb_ref[...], axis=0, keepdims=True)
    o_ref[...] = xb + LAMDA * (xb * a - c)


def _dgn_prep(p):
    W, b = p["lin"]["W"], p["lin"]["b"]
    wnt = jnp.zeros((HID, 16), jnp.float32).at[:, :GROUPS].set(W.T)
    bn16 = jnp.full((16,), NEG, jnp.float32).at[:GROUPS].set(b)
    bw = jnp.zeros((16, HID), jnp.float32).at[:GROUPS].set(
        p["bn_w"].reshape(GROUPS, HID))
    bb = jnp.zeros((16, HID), jnp.float32).at[:GROUPS].set(
        p["bn_b"].reshape(GROUPS, HID))
    return wnt, _pad8(bn16), bw, bb


def _dgn_node(x, p):
    wnt, bn16, bw, bb = _dgn_prep(p)
    blk = _NODE_BLK
    grid = N // blk
    full = lambda s: pl.BlockSpec(s, lambda i: (0, 0))
    m, q = pl.pallas_call(
        _dgn_stats_body,
        grid=(grid,),
        in_specs=[pl.BlockSpec((blk, HID), lambda i: (i, 0)),
                  full((HID, 16)), full((8, 16))],
        out_specs=[full((16, HID)), full((16, HID))],
        out_shape=[jax.ShapeDtypeStruct((16, HID), jnp.float32)] * 2,
    )(x, wnt, bn16)
    return pl.pallas_call(
        functools.partial(_dgn_apply_body, float(N)),
        grid=(grid,),
        in_specs=[pl.BlockSpec((blk, HID), lambda i: (i, 0)),
                  full((HID, 16)), full((8, 16)), full((16, HID)),
                  full((16, HID)), full((16, HID)), full((16, HID))],
        out_specs=pl.BlockSpec((blk, HID), lambda i: (i, 0)),
        out_shape=jax.ShapeDtypeStruct((N, HID), jnp.float32),
    )(x, wnt, bn16, m, q, bw, bb)


def _dgn_graph_body(x_ref, wnt_ref, bn_ref, bw_ref, bb_ref, o_ref):
    xb = x_ref[...]
    s = _softmax_s(xb, wnt_ref[...], bn_ref[...])
    mp = lax.dot_general(s, xb, (((0,), (0,)), ((), ())),
                         preferred_element_type=jnp.float32)
    qp = lax.dot_general(s * s, xb * xb, (((0,), (0,)), ((), ())),
                         preferred_element_type=jnp.float32)
    mean = mp * (1.0 / B)
    var = qp * (1.0 / B) - mean * mean
    rstd = lax.rsqrt(var + EPS)
    wsc = bw_ref[...] * rstd
    a = jnp.dot(s, wsc, preferred_element_type=jnp.float32)
    c = jnp.sum(mean * wsc - bb_ref[...], axis=0, keepdims=True)
    o_ref[...] = xb + LAMDA * (xb * a - c)


def _dgn_graph(x, p):
    wnt, bn16, bw, bb = _dgn_prep(p)
    return pl.pallas_call(
        _dgn_graph_body,
        out_shape=jax.ShapeDtypeStruct((B, HID), jnp.float32),
    )(x, wnt, bn16, bw, bb)


def _pool_body(nblocks, bf_ref, x_ref, o_ref):
    i = pl.program_id(0)
    iota = lax.broadcasted_iota(jnp.int32, (1, B), 1).astype(jnp.float32)
    onehot = (bf_ref[...][:, 0:1] == iota).astype(jnp.float32)
    part = lax.dot_general(onehot, x_ref[...], (((0,), (0,)), ((), ())),
                           preferred_element_type=jnp.float32)

    @pl.when(i == 0)
    def _():
        o_ref[...] = jnp.zeros_like(o_ref)

    o_ref[...] += part

    @pl.when(i == nblocks - 1)
    def _():
        o_ref[...] = jnp.maximum(o_ref[...], 0.0)


def _pool(x, batchf):
    blk = _NODE_BLK
    grid = N // blk
    return pl.pallas_call(
        functools.partial(_pool_body, grid),
        grid=(grid,),
        in_specs=[pl.BlockSpec((blk, 8), lambda i: (i, 0)),
                  pl.BlockSpec((blk, HID), lambda i: (i, 0))],
        out_specs=pl.BlockSpec((B, HID), lambda i: (0, 0)),
        out_shape=jax.ShapeDtypeStruct((B, HID), jnp.float32),
    )(batchf, x)


def _ggat_alpha(xlb, bf, xr, att):
    iota = lax.broadcasted_iota(jnp.int32, (1, B), 1).astype(jnp.float32)
    onehot = (bf[:, 0:1] == iota).astype(jnp.float32)
    e = xlb + jnp.dot(onehot, xr, preferred_element_type=jnp.float32)
    e = jnp.where(e > 0, e, 0.01 * e)
    alpha = jnp.dot(e, att, preferred_element_type=jnp.float32)
    return onehot, alpha


def _ggat1_body(nblocks, xl_ref, bf_ref, xr_ref, att_ref, amax_ref):
    i = pl.program_id(0)
    onehot, alpha = _ggat_alpha(xl_ref[...], bf_ref[...], xr_ref[...],
                                att_ref[...][:, 0:1])
    masked = jnp.where(onehot > 0, alpha, NEG)
    pmax = jnp.max(masked, axis=0, keepdims=True)

    @pl.when(i == 0)
    def _():
        amax_ref[...] = jnp.full_like(amax_ref, NEG)

    amax_ref[...] = jnp.maximum(amax_ref[...], jnp.broadcast_to(pmax, (8, B)))


def _ggat2_body(xl_ref, bf_ref, xr_ref, att_ref, amax_ref, num_ref, den_ref):
    i = pl.program_id(0)
    onehot, alpha = _ggat_alpha(xl_ref[...], bf_ref[...], xr_ref[...],
                                att_ref[...][:, 0:1])
    am = amax_ref[...][0:1, :]
    am = jnp.where(am < -1e29, 0.0, am)
    amrow = jnp.sum(onehot * am, axis=1, keepdims=True)
    ex = jnp.exp(alpha - amrow)
    np_ = lax.dot_general(onehot, ex * xl_ref[...], (((0,), (0,)), ((), ())),
                          preferred_element_type=jnp.float32)
    dp = lax.dot_general(onehot, jnp.broadcast_to(ex, ex.shape[:1] + (8,)),
                         (((0,), (0,)), ((), ())),
                         preferred_element_type=jnp.float32)

    @pl.when(i == 0)
    def _():
        num_ref[...] = jnp.zeros_like(num_ref)
        den_ref[...] = jnp.zeros_like(den_ref)

    num_ref[...] += np_
    den_ref[...] += dp


def _ggat(xl, batchf, xr, att):
    blk = _NODE_BLK
    grid = N // blk
    full = lambda s: pl.BlockSpec(s, lambda i: (0, 0))
    att2 = jnp.broadcast_to(att[:, None], (HID, 8))
    amax = pl.pallas_call(
        functools.partial(_ggat1_body, grid),
        grid=(grid,),
        in_specs=[pl.BlockSpec((blk, HID), lambda i: (i, 0)),
                  pl.BlockSpec((blk, 8), lambda i: (i, 0)),
                  full((B, HID)), full((HID, 8))],
        out_specs=full((8, B)),
        out_shape=jax.ShapeDtypeStruct((8, B), jnp.float32),
    )(xl, batchf, xr, att2)
    num, den = pl.pallas_call(
        _ggat2_body,
        grid=(grid,),
        in_specs=[pl.BlockSpec((blk, HID), lambda i: (i, 0)),
                  pl.BlockSpec((blk, 8), lambda i: (i, 0)),
                  full((B, HID)), full((HID, 8)), full((8, B))],
        out_specs=[full((B, HID)), full((B, 8))],
        out_shape=[jax.ShapeDtypeStruct((B, HID), jnp.float32),
                   jax.ShapeDtypeStruct((B, 8), jnp.float32)],
    )(xl, batchf, xr, att2, amax)
    return num, den


_NC, _NS, _L = 2, 16, 16
_NW = _NC * _NS
_EPT = E // _NW
_K = 80
_NCHUNK = _EPT // _K
_CPY = 640
_CSTEP = 624

_MESH = plsc.VectorSubcoreMesh(core_axis_name="c", subcore_axis_name="s")


def _lane_iota():
    return lax.broadcasted_iota(jnp.int32, (_L,), 0)


def _shuf(v, idx):
    dnums = lax.GatherDimensionNumbers(
        offset_dims=(), collapsed_slice_dims=(0,), start_index_map=(0,))
    return lax.gather(v, idx[:, None], dnums, slice_sizes=(1,),
                      mode=lax.GatherScatterMode.PROMISE_IN_BOUNDS)


def _lane_sum(v):
    lane = _lane_iota()
    for s in (8, 4, 2, 1):
        v = v + _shuf(v, lane ^ s)
    return v


def _lane_max(v):
    lane = _lane_iota()
    for s in (8, 4, 2, 1):
        v = jnp.maximum(v, _shuf(v, lane ^ s))
    return v


_SBK = 5
_SBE = _SBK * _K
_NSB = _NCHUNK // _SBK


def _alpha_compute(gsrc, gdst, sev, attv, alv, off, mx):
    for g in range(_K // _L):
        av = jnp.zeros((_L,), jnp.float32)
        for j in range(_L):
            i = g * _L + j
            acc = jnp.zeros((_L,), jnp.float32)
            for f in range(HID // _L):
                v = (gsrc[i, pl.ds(f * _L, _L)]
                     + gdst[i, pl.ds(HID + f * _L, _L)]
                     + sev[i, pl.ds(f * _L, _L)])
                v = jnp.maximum(v, 0.0) + 0.01 * jnp.minimum(v, 0.0)
                acc = acc + v * attv[pl.ds(f * _L, _L)]
            a = _lane_sum(acc)
            av = jnp.where(_lane_iota() == j, a, av)
        alv[pl.ds(off + g * _L, _L)] = av
        mx = jnp.maximum(mx, av)
    return mx


def _sc_alpha(xlr, se, src, dst, att):
    @functools.partial(
        pl.kernel,
        out_type=[jax.ShapeDtypeStruct((E,), jnp.float32),
                  jax.ShapeDtypeStruct((_NW, 16), jnp.float32)],
        mesh=_MESH,
        scratch_types=[
            pltpu.VMEM((_SBE,), jnp.int32),
            pltpu.VMEM((_SBE,), jnp.int32),
            pltpu.VMEM((_SBE,), jnp.float32),
            pltpu.VMEM((_K, 2 * HID), jnp.float32),
            pltpu.VMEM((_K, 2 * HID), jnp.float32),
            pltpu.VMEM((_K, 2 * HID), jnp.float32),
            pltpu.VMEM((_K, 2 * HID), jnp.float32),
            pltpu.VMEM((_K, HID), jnp.float32),
            pltpu.VMEM((_K, HID), jnp.float32),
            pltpu.VMEM((HID,), jnp.float32),
            pltpu.VMEM((16,), jnp.float32),
            pltpu.SemaphoreType.DMA,
            pltpu.SemaphoreType.DMA,
            pltpu.SemaphoreType.DMA,
            pltpu.SemaphoreType.DMA,
            pltpu.SemaphoreType.DMA,
            pltpu.SemaphoreType.DMA,
            pltpu.SemaphoreType.DMA,
        ],
    )
    def body(xlr_h, se_h, src_h, dst_h, att_h, alpha_h, tmax_h,
             srcb, dstb, albuf, gsrc0, gsrc1, gdst0, gdst1,
             sev0, sev1, attv, mxv, six, s0a, s0b, s1a, s1b, sse0, sse1):
        cid = lax.axis_index("c")
        sid = lax.axis_index("s")
        wid = sid * _NC + cid
        pltpu.sync_copy(att_h, attv)
        gsrc = (gsrc0, gsrc1)
        gdst = (gdst0, gdst1)
        sev = (sev0, sev1)
        sg = ((s0a, s0b), (s1a, s1b))
        sse = (sse0, sse1)

        def sblock(sb, mx):
            base = wid * _EPT + sb * _SBE
            ix1 = pltpu.async_copy(src_h.at[pl.ds(base, _SBE)], srcb, six)
            ix2 = pltpu.async_copy(dst_h.at[pl.ds(base, _SBE)], dstb, six)
            secp = {}
            for j in range(2):
                secp[j] = pltpu.async_copy(
                    se_h.at[pl.ds(base + j * _K, _K)], sev[j], sse[j])
            ix1.wait()
            ix2.wait()
            gcp = {}
            for j in range(2):
                b = j % 2
                gcp[j] = (
                    pltpu.async_copy(
                        xlr_h.at[srcb.at[pl.ds(j * _K, _K)]], gsrc[b],
                        sg[b][0]),
                    pltpu.async_copy(
                        xlr_h.at[dstb.at[pl.ds(j * _K, _K)]], gdst[b],
                        sg[b][1]))
            for j in range(_SBK):
                b = j % 2
                gcp[j][0].wait()
                gcp[j][1].wait()
                secp[j].wait()
                mx = _alpha_compute(gsrc[b], gdst[b], sev[b], attv,
                                    albuf, j * _K, mx)
                if j + 2 < _SBK:
                    gcp[j + 2] = (
                        pltpu.async_copy(
                            xlr_h.at[srcb.at[pl.ds((j + 2) * _K, _K)]],
                            gsrc[b], sg[b][0]),
                        pltpu.async_copy(
                            xlr_h.at[dstb.at[pl.ds((j + 2) * _K, _K)]],
                            gdst[b], sg[b][1]))
                    secp[j + 2] = pltpu.async_copy(
                        se_h.at[pl.ds(base + (j + 2) * _K, _K)], sev[b],
                        sse[b])
            pltpu.sync_copy(albuf, alpha_h.at[pl.ds(base, _SBE)])
            return mx

        mx = lax.fori_loop(0, _NSB, sblock,
                           jnp.full((_L,), NEG, jnp.float32))

        mxv[...] = mx
        pltpu.sync_copy(mxv, tmax_h.at[wid])

    return body(xlr, se, src, dst, att)


_AW = 2 * HID


def _scatter_compute(xlv, alv, off, gmax, scn, lane):
    for g in range(_K // _L):
        ex16 = jnp.exp(alv[pl.ds(off + g * _L, _L)] - gmax)
        for j in range(_L):
            i = g * _L + j
            exb = _shuf(ex16, jnp.full((_L,), j, jnp.int32))
            for f in range(HID // _L):
                scn[i, pl.ds(f * _L, _L)] = xlv[i, pl.ds(f * _L, _L)] * exb
            scn[i, pl.ds(HID, _L)] = jnp.where(lane == 0, exb, 0.0)


def _sc_scatter(xlr, src, dst, alpha, tmax, zn):
    @functools.partial(
        pl.kernel,
        out_type=jax.ShapeDtypeStruct((2 * N, _AW), jnp.float32),
        mesh=_MESH,
        scratch_types=[
            pltpu.VMEM((_SBE,), jnp.int32),
            pltpu.VMEM((_SBE,), jnp.float32),
            pltpu.VMEM((_K,), jnp.int32),
            pltpu.VMEM((_K,), jnp.int32),
            pltpu.VMEM((_K,), jnp.int32),
            pltpu.VMEM((_K,), jnp.int32),
            pltpu.VMEM((_K,), jnp.int32),
            pltpu.VMEM((_K, 2 * HID), jnp.float32),
            pltpu.VMEM((_K, 2 * HID), jnp.float32),
            pltpu.VMEM((_K, _AW), jnp.float32),
            pltpu.VMEM((_K, _AW), jnp.float32),
            pltpu.VMEM((_NW, 16), jnp.float32),
            pltpu.VMEM_SHARED((N, _AW), jnp.float32),
            pltpu.SemaphoreType.DMA,
            pltpu.SemaphoreType.DMA,
            pltpu.SemaphoreType.DMA,
            pltpu.SemaphoreType.DMA,
            pltpu.SemaphoreType.DMA,
        ],
    )
    def body(xlr_h, src_h, dst_h, alpha_h, tmax_h, zn_h, acc_h,
             srcb, albuf, dstv0, dstv1, dstv2, dstv3, dstv4, xlv0, xlv1,
             scn0, scn1, tmaxv, acc_s, six, sg0, sg1, sw0, sw1):
        cid = lax.axis_index("c")
        sid = lax.axis_index("s")
        wid = sid * _NC + cid

        pltpu.sync_copy(zn_h, scn0)
        for k in range(_CPY // _K):
            pltpu.sync_copy(scn0, acc_s.at[pl.ds(sid * _CSTEP + k * _K, _K)])

        zero = jnp.zeros((_L,), jnp.float32)
        for i in range(_K):
            for f in range(HID + _L, 2 * HID, _L):
                scn0[i, pl.ds(f, _L)] = zero
                scn1[i, pl.ds(f, _L)] = zero

        pltpu.sync_copy(tmax_h, tmaxv)
        mm = tmaxv[0, :]
        for r in range(1, _NW):
            mm = jnp.maximum(mm, tmaxv[r, :])
        gmax = _lane_max(mm)

        plsc.subcore_barrier()
        lane = _lane_iota()
        dstv = (dstv0, dstv1, dstv2, dstv3, dstv4)
        xlv = (xlv0, xlv1)
        scn = (scn0, scn1)
        sg = (sg0, sg1)
        sw = (sw0, sw1)

        def sblock(sb, carry):
            base = wid * _EPT + sb * _SBE
            ixcp = [pltpu.async_copy(src_h.at[pl.ds(base, _SBE)], srcb, six),
                    pltpu.async_copy(alpha_h.at[pl.ds(base, _SBE)], albuf,
                                     six)]
            for j in range(_SBK):
                ixcp.append(pltpu.async_copy(
                    dst_h.at[pl.ds(base + j * _K, _K)], dstv[j], six))
            for cp in ixcp:
                cp.wait()
            gcp = {}
            for j in range(2):
                gcp[j] = pltpu.async_copy(
                    xlr_h.at[srcb.at[pl.ds(j * _K, _K)]], xlv[j], sg[j])
            scp = {}
            for j in range(_SBK):
                b = j % 2
                gcp[j].wait()
                if j >= 2:
                    scp[j - 2].wait()
                _scatter_compute(xlv[b], albuf, j * _K, gmax, scn[b], lane)
                scp[j] = pltpu.async_copy(scn[b], acc_s.at[dstv[j]], sw[b],
                                          add=True)
                if j + 2 < _SBK:
                    gcp[j + 2] = pltpu.async_copy(
                        xlr_h.at[srcb.at[pl.ds((j + 2) * _K, _K)]], xlv[b],
                        sg[b])
            scp[_SBK - 2].wait()
            scp[_SBK - 1].wait()
            return carry

        lax.fori_loop(0, _NSB, sblock, jnp.int32(0))

        plsc.subcore_barrier()

        for k in range(_CPY // _K):
            off = sid * _CSTEP + k * _K
            pltpu.sync_copy(acc_s.at[pl.ds(off, _K)], scn0)
            pltpu.sync_copy(scn0, acc_h.at[pl.ds(cid * N + off, _K)])

    return body(xlr, src, dst, alpha, tmax, zn)


def _edge_phase(xlr, se, src, dst, att, zn):
    alpha, tmax = _sc_alpha(xlr, se, src, dst, att)
    acc = _sc_scatter(xlr, src, dst, alpha, tmax, zn)
    return (acc[:N, :HID], acc[N:, :HID],
            acc[:N, HID:HID + 16], acc[N:, HID:HID + 16])


def kernel(x, edge_index, edge_attr, batch, params):
    src, dst = edge_index[0], edge_index[1]
    batchf = jnp.broadcast_to(batch.astype(jnp.float32)[:, None], (N, 8))
    zn = jnp.zeros((_K, _AW), jnp.float32)

    x = _linear(x, params["pre_node"]["W"], params["pre_node"]["b"], "silu",
                _NODE_BLK)
    ea = _linear(edge_attr, params["pre_edge"]["W"], params["pre_edge"]["b"],
                 "silu", _EDGE_BLK)

    for name in ("layer0", "layer1"):
        lp = params[name]
        cp = lp["conv"]
        se = _linear(ea, cp["lin_e_W"], jnp.zeros((HID,), jnp.float32), None,
                     _EDGE_BLK)
        W_lr = jnp.concatenate([cp["lin_l"]["W"], cp["lin_r"]["W"]], axis=0)
        b_lr = jnp.concatenate([cp["lin_l"]["b"], cp["lin_r"]["b"]])
        for _ in range(2):
            xlr = _linear(x, W_lr, b_lr, None, _NODE_BLK)
            n0, n1, d0, d1 = _edge_phase(xlr, se, src, dst, cp["att"], zn)
            x = _gru_node(n0, n1, d0, d1, cp["bias"], x, lp["gru"])
        x = _dgn_node(x, lp["norm"])

    out = _pool(x, batchf)
    gp = params["gconv"]
    xl_g = _linear(x, gp["lin_l"]["W"], gp["lin_l"]["b"], None, _NODE_BLK)
    for _ in range(2):
        xr_g = _linear(out, gp["lin_r"]["W"], gp["lin_r"]["b"], None, B)
        num, den = _ggat(xl_g, batchf, xr_g, gp["att"])
        out = _gru_graph(num, den, gp["bias"], out, params["ggru"])
    y = _dgn_graph(out, params["gnorm"])
    y = _linear(y, params["post0"]["W"], params["post0"]["b"], "silu", B)
    y = _linear(y, params["post1"]["W"], params["post1"]["b"], "silu", B)
    return _linear(y, params["out"]["W"], params["out"]["b"], None, B)
